# Initial kernel scaffold; baseline (speedup 1.0000x reference)
import jax
import jax.numpy as jnp
from jax.experimental import pallas as pl


def kernel(x, adj, params):
    raise NotImplementedError("write your pallas kernel here")


# speedup vs baseline: 1.9528x; 1.9528x over previous
import functools

import jax
import jax.numpy as jnp
from jax import lax
from jax.experimental import pallas as pl
from jax.experimental.pallas import tpu as pltpu
from jax.experimental.pallas import tpu_sc as plsc

N = 10000
E = 320000
NC, NS, L = 2, 16, 16
NW = NC * NS
NPW = (N + NW - 1) // NW
NPAD = NW * NPW
G = 128
CAP = G + L
CHUNK = 4000
NEG = -3.0e38

BN = 1000


def _norm_body(x_ref, o_ref):
    x = x_ref[...]
    ss = jnp.sum(x * x, axis=1, keepdims=True)
    nrm = jnp.maximum(jnp.sqrt(ss), 1e-12)
    o_ref[...] = x / nrm


def _normalize(x):
    return pl.pallas_call(
        _norm_body,
        out_shape=jax.ShapeDtypeStruct((N, 128), jnp.float32),
        grid=(N // BN,),
        in_specs=[pl.BlockSpec((BN, 128), lambda i: (i, 0))],
        out_specs=pl.BlockSpec((BN, 128), lambda i: (i, 0)),
    )(x)


def _pool_body(h_ref, w_ref, b_ref, o_ref):
    acc = jnp.dot(h_ref[...], w_ref[...], preferred_element_type=jnp.float32)
    o_ref[...] = jnp.maximum(acc + b_ref[...], 0.0)


def _pool(h, w, b):
    dpi = h.shape[1]
    return pl.pallas_call(
        _pool_body,
        out_shape=jax.ShapeDtypeStruct((N, dpi), jnp.float32),
        grid=(N // BN,),
        in_specs=[
            pl.BlockSpec((BN, dpi), lambda i: (i, 0)),
            pl.BlockSpec((dpi, dpi), lambda i: (0, 0)),
            pl.BlockSpec((1, dpi), lambda i: (0, 0)),
        ],
        out_specs=pl.BlockSpec((BN, dpi), lambda i: (i, 0)),
    )(h, w, b)


def _combine_body(h_ref, a_ref, ws_ref, wn_ref, b_ref, o_ref, *, relu):
    a = a_ref[...]
    a = jnp.where(a < -1e30, 0.0, a)
    acc = jnp.dot(h_ref[...], ws_ref[...], preferred_element_type=jnp.float32)
    acc += jnp.dot(a, wn_ref[...], preferred_element_type=jnp.float32)
    acc += b_ref[...]
    if relu:
        acc = jnp.maximum(acc, 0.0)
    o_ref[...] = acc


def _combine(h, agg, ws, wn, b, relu):
    dpi = h.shape[1]
    dpo = ws.shape[1]
    return pl.pallas_call(
        functools.partial(_combine_body, relu=relu),
        out_shape=jax.ShapeDtypeStruct((N, dpo), jnp.float32),
        grid=(N // BN,),
        in_specs=[
            pl.BlockSpec((BN, dpi), lambda i: (i, 0)),
            pl.BlockSpec((BN, dpi), lambda i: (i, 0)),
            pl.BlockSpec((dpi, dpo), lambda i: (0, 0)),
            pl.BlockSpec((dpi, dpo), lambda i: (0, 0)),
            pl.BlockSpec((1, dpo), lambda i: (0, 0)),
        ],
        out_specs=pl.BlockSpec((BN, dpo), lambda i: (i, 0)),
    )(h, agg, ws, wn, b)


_MESH = plsc.VectorSubcoreMesh(
    core_axis_name="c", subcore_axis_name="s", num_cores=NC, num_subcores=NS)
_SC_PARAMS = pltpu.CompilerParams(needs_layout_passes=False)


def _wid():
    return lax.axis_index("s") * NC + lax.axis_index("c")


def _partition_body(src_hbm, dst_hbm, srcp_hbm, locp_hbm, cnt_hbm,
                    src_v, dst_v, sel_s, sel_l, cnt_v, sem):
    wid = _wid()
    lo = wid * NPW
    hi = lo + NPW

    for i in range(CAP // L):
        sel_s[pl.ds(i * L, L)] = jnp.zeros((L,), jnp.int32)
        sel_l[pl.ds(i * L, L)] = jnp.zeros((L,), jnp.int32)

    def grp_body(g, carry):
        nsel, nfired = carry
        d16 = dst_v[pl.ds(g * L, L)]
        s16 = src_v[pl.ds(g * L, L)]
        m = (d16 >= lo) & (d16 < hi)
        mi = m.astype(jnp.int32)
        cum = plsc.cumsum(mi)
        idx = nsel + cum - 1
        plsc.store_scatter(sel_s, [idx], s16, mask=m)
        plsc.store_scatter(sel_l, [idx], d16 - lo, mask=m)
        nsel = nsel + jnp.max(cum)

        full = nsel >= G

        @pl.when(full)
        def _fire():
            pltpu.sync_copy(sel_s.at[pl.ds(0, G)],
                            srcp_hbm.at[wid, pl.ds(nfired * G, G)])
            pltpu.sync_copy(sel_l.at[pl.ds(0, G)],
                            locp_hbm.at[wid, pl.ds(nfired * G, G)])
            sel_s[pl.ds(0, L)] = sel_s[pl.ds(G, L)]
            sel_l[pl.ds(0, L)] = sel_l[pl.ds(G, L)]

        nsel = jnp.where(full, nsel - G, nsel)
        nfired = jnp.where(full, nfired + 1, nfired)
        return nsel, nfired

    def chunk_body(ci, carry):
        pltpu.sync_copy(src_hbm.at[pl.ds(ci * CHUNK, CHUNK)], src_v)
        pltpu.sync_copy(dst_hbm.at[pl.ds(ci * CHUNK, CHUNK)], dst_v)
        return lax.fori_loop(0, CHUNK // L, grp_body, carry)

    nsel, nfired = lax.fori_loop(0, E // CHUNK, chunk_body, (0, 0))

    @pl.when(nsel > 0)
    def _final():
        pltpu.sync_copy(sel_s.at[pl.ds(0, G)],
                        srcp_hbm.at[wid, pl.ds(nfired * G, G)])
        pltpu.sync_copy(sel_l.at[pl.ds(0, G)],
                        locp_hbm.at[wid, pl.ds(nfired * G, G)])

    count = nfired * G + nsel
    cnt_v[pl.ds(0, L)] = jnp.full((L,), 1, jnp.int32) * count
    pltpu.sync_copy(cnt_v, cnt_hbm.at[wid])


_partition = pl.kernel(
    _partition_body,
    out_type=[
        jax.ShapeDtypeStruct((NW, E), jnp.int32),
        jax.ShapeDtypeStruct((NW, E), jnp.int32),
        jax.ShapeDtypeStruct((NW, L), jnp.int32),
    ],
    mesh=_MESH,
    compiler_params=_SC_PARAMS,
    scratch_types=[
        pltpu.VMEM((CHUNK,), jnp.int32),
        pltpu.VMEM((CHUNK,), jnp.int32),
        pltpu.VMEM((CAP,), jnp.int32),
        pltpu.VMEM((CAP,), jnp.int32),
        pltpu.VMEM((L,), jnp.int32),
        pltpu.SemaphoreType.DMA,
    ],
)


def _segmax_body(hp_hbm, srcp_hbm, locp_hbm, cnt_hbm, out_hbm,
                 agg_v, srcb_v, locb_v, cnt_v, rows_v, sem, *, dpad):
    wid = _wid()

    pltpu.sync_copy(cnt_hbm.at[wid], cnt_v)
    count = cnt_v[pl.ds(0, L)][0]

    def init_body(i, _):
        agg_v[pl.ds(i * L, L)] = jnp.full((L,), NEG, jnp.float32)
        return 0
    lax.fori_loop(0, NPW * dpad // L, init_body, 0)

    def batch_body(b, _):
        base = b * G
        pltpu.sync_copy(srcp_hbm.at[wid, pl.ds(base, G)], srcb_v)
        pltpu.sync_copy(locp_hbm.at[wid, pl.ds(base, G)], locb_v.at[pl.ds(0, G)])
        pltpu.async_copy(hp_hbm.at[srcb_v], rows_v, sem).wait()
        nd = jnp.minimum(G, count - base)

        def edge_body(j, _):
            loc = locb_v[pl.ds(j, L)][0]
            rb = loc * dpad
            for dblk in range(dpad // L):
                off = dblk * L
                agg_v[pl.ds(rb + off, L)] = jnp.maximum(
                    agg_v[pl.ds(rb + off, L)], rows_v[j, pl.ds(off, L)])
            return 0

        lax.fori_loop(0, nd, edge_body, 0)
        return 0

    nb = (count + G - 1) // G
    lax.fori_loop(0, nb, batch_body, 0)
    pltpu.sync_copy(agg_v, out_hbm.at[pl.ds(wid * NPW * dpad, NPW * dpad)])


@functools.lru_cache(maxsize=None)
def _make_segmax(dpad):
    return pl.kernel(
        functools.partial(_segmax_body, dpad=dpad),
        out_type=jax.ShapeDtypeStruct((NPAD * dpad,), jnp.float32),
        mesh=_MESH,
        compiler_params=_SC_PARAMS,
        scratch_types=[
            pltpu.VMEM((NPW * dpad,), jnp.float32),
            pltpu.VMEM((G,), jnp.int32),
            pltpu.VMEM((G + L,), jnp.int32),
            pltpu.VMEM((L,), jnp.int32),
            pltpu.VMEM((G, dpad), jnp.float32),
            pltpu.SemaphoreType.DMA,
        ],
    )


def _pad2(w, r, c):
    return jnp.pad(w, ((0, r - w.shape[0]), (0, c - w.shape[1])))


def kernel(x, adj, params):
    src = adj[0]
    dst = adj[1]

    dpi_l = [128, 128, 128, 128, 128]
    dpo_l = [128, 128, 128, 128, 128]

    h = _normalize(x)
    srcp, locp, cnts = _partition(src, dst)

    for i in range(5):
        dpi, dpo = dpi_l[i], dpo_l[i]
        wp = _pad2(params['Wp%d' % i], dpi, dpi)
        bp = jnp.pad(params['bp%d' % i], (0, dpi - params['bp%d' % i].shape[0]))
        ws = _pad2(params['Ws%d' % i], dpi, dpo)
        wn = _pad2(params['Wn%d' % i], dpi, dpo)
        b = jnp.pad(params['b%d' % i], (0, dpo - params['b%d' % i].shape[0]))

        hp = _pool(h, wp, bp.reshape(1, dpi))
        agg_flat = _make_segmax(dpi)(hp, srcp, locp, cnts)
        agg = agg_flat.reshape(NPAD, dpi)[:N]
        h = _combine(h, agg, ws, wn, b.reshape(1, dpo), relu=(i < 4))

    return h


# speedup vs baseline: 2.5314x; 1.2963x over previous
import functools

import jax
import jax.numpy as jnp
from jax import lax
from jax.experimental import pallas as pl
from jax.experimental.pallas import tpu as pltpu
from jax.experimental.pallas import tpu_sc as plsc

N = 10000
E = 320000
NC, NS, L = 2, 16, 16
NW = NC * NS
NPW = (N + NW - 1) // NW
NPAD = NW * NPW
G = 128
CAP = G + L
CHUNK = 6400
NEG = -3.0e38

BN = 1000


def _norm_body(x_ref, o_ref):
    x = x_ref[...]
    ss = jnp.sum(x * x, axis=1, keepdims=True)
    nrm = jnp.maximum(jnp.sqrt(ss), 1e-12)
    o_ref[...] = x / nrm


def _normalize(x):
    return pl.pallas_call(
        _norm_body,
        out_shape=jax.ShapeDtypeStruct((N, 128), jnp.float32),
        grid=(N // BN,),
        in_specs=[pl.BlockSpec((BN, 128), lambda i: (i, 0))],
        out_specs=pl.BlockSpec((BN, 128), lambda i: (i, 0)),
    )(x)


def _pool_body(h_ref, w_ref, b_ref, o_ref):
    acc = jnp.dot(h_ref[...], w_ref[...], preferred_element_type=jnp.float32)
    o_ref[...] = jnp.maximum(acc + b_ref[...], 0.0)


def _pool(h, w, b):
    dpi = h.shape[1]
    return pl.pallas_call(
        _pool_body,
        out_shape=jax.ShapeDtypeStruct((N, dpi), jnp.float32),
        grid=(N // BN,),
        in_specs=[
            pl.BlockSpec((BN, dpi), lambda i: (i, 0)),
            pl.BlockSpec((dpi, dpi), lambda i: (0, 0)),
            pl.BlockSpec((1, dpi), lambda i: (0, 0)),
        ],
        out_specs=pl.BlockSpec((BN, dpi), lambda i: (i, 0)),
    )(h, w, b)


def _combine_body(h_ref, a_ref, ws_ref, wn_ref, b_ref, o_ref, *, relu):
    a = a_ref[...]
    a = jnp.where(a < -1e30, 0.0, a)
    acc = jnp.dot(h_ref[...], ws_ref[...], preferred_element_type=jnp.float32)
    acc += jnp.dot(a, wn_ref[...], preferred_element_type=jnp.float32)
    acc += b_ref[...]
    if relu:
        acc = jnp.maximum(acc, 0.0)
    o_ref[...] = acc


def _combine(h, agg, ws, wn, b, relu):
    dpi = h.shape[1]
    dpo = ws.shape[1]
    return pl.pallas_call(
        functools.partial(_combine_body, relu=relu),
        out_shape=jax.ShapeDtypeStruct((N, dpo), jnp.float32),
        grid=(N // BN,),
        in_specs=[
            pl.BlockSpec((BN, dpi), lambda i: (i, 0)),
            pl.BlockSpec((BN, dpi), lambda i: (i, 0)),
            pl.BlockSpec((dpi, dpo), lambda i: (0, 0)),
            pl.BlockSpec((dpi, dpo), lambda i: (0, 0)),
            pl.BlockSpec((1, dpo), lambda i: (0, 0)),
        ],
        out_specs=pl.BlockSpec((BN, dpo), lambda i: (i, 0)),
    )(h, agg, ws, wn, b)


_MESH = plsc.VectorSubcoreMesh(
    core_axis_name="c", subcore_axis_name="s", num_cores=NC, num_subcores=NS)
_SC_PARAMS = pltpu.CompilerParams(needs_layout_passes=False)


def _wid():
    return lax.axis_index("s") * NC + lax.axis_index("c")


NCHUNK = E // CHUNK


def _partition_body(adj_hbm, part_hbm, cnt_hbm,
                    ad0_v, ad1_v, sel_p, cnt_v, sem0, sem1):
    wid = _wid()
    lo = wid * NPW
    hi = lo + NPW
    ad_v = (ad0_v, ad1_v)
    sems = (sem0, sem1)

    for i in range(CAP // L):
        sel_p[pl.ds(i * L, L)] = jnp.zeros((L,), jnp.int32)

    def grp_body(ad, g, carry):
        nsel, nfired = carry
        s16 = ad[0, pl.ds(g * L, L)]
        d16 = ad[1, pl.ds(g * L, L)]
        m = (d16 >= lo) & (d16 < hi)
        cum = plsc.cumsum(m.astype(jnp.int32))
        idx = nsel + cum - 1
        pack = s16 | ((d16 - lo) << 16)
        plsc.store_scatter(sel_p, [idx], pack, mask=m)
        nsel = nsel + cum[L - 1]

        full = nsel >= G

        @pl.when(full)
        def _fire():
            pltpu.sync_copy(sel_p.at[pl.ds(0, G)],
                            part_hbm.at[wid, pl.ds(nfired * G, G)])
            sel_p[pl.ds(0, L)] = sel_p[pl.ds(G, L)]

        nsel = jnp.where(full, nsel - G, nsel)
        nfired = jnp.where(full, nfired + 1, nfired)
        return nsel, nfired

    pltpu.async_copy(adj_hbm.at[:, pl.ds(0, CHUNK)], ad0_v, sem0)
    pltpu.async_copy(adj_hbm.at[:, pl.ds(CHUNK, CHUNK)], ad1_v, sem1)

    def pair_body(i, carry):
        for p in range(2):
            b = 2 * i + p
            pltpu.make_async_copy(
                adj_hbm.at[:, pl.ds(b * CHUNK, CHUNK)], ad_v[p], sems[p]
            ).wait()
            carry = lax.fori_loop(
                0, CHUNK // L, functools.partial(grp_body, ad_v[p]), carry)

            @pl.when(b + 2 < NCHUNK)
            def _refill():
                pltpu.async_copy(
                    adj_hbm.at[:, pl.ds((b + 2) * CHUNK, CHUNK)],
                    ad_v[p], sems[p])
        return carry

    assert NCHUNK % 2 == 0
    nsel, nfired = lax.fori_loop(0, NCHUNK // 2, pair_body, (0, 0))

    @pl.when(nsel > 0)
    def _final():
        pltpu.sync_copy(sel_p.at[pl.ds(0, G)],
                        part_hbm.at[wid, pl.ds(nfired * G, G)])

    count = nfired * G + nsel
    cnt_v[pl.ds(0, L)] = jnp.full((L,), 1, jnp.int32) * count
    pltpu.sync_copy(cnt_v, cnt_hbm.at[wid])


_partition = pl.kernel(
    _partition_body,
    out_type=[
        jax.ShapeDtypeStruct((NW, E), jnp.int32),
        jax.ShapeDtypeStruct((NW, L), jnp.int32),
    ],
    mesh=_MESH,
    compiler_params=_SC_PARAMS,
    scratch_types=[
        pltpu.VMEM((2, CHUNK), jnp.int32),
        pltpu.VMEM((2, CHUNK), jnp.int32),
        pltpu.VMEM((CAP,), jnp.int32),
        pltpu.VMEM((L,), jnp.int32),
        pltpu.SemaphoreType.DMA,
        pltpu.SemaphoreType.DMA,
    ],
)


def _segmax_body(hp_hbm, part_hbm, cnt_hbm, out_hbm,
                 agg_v, pk0_v, pk1_v, sb0_v, sb1_v, r0_v, r1_v, cnt_v,
                 sp0, sp1, sg0, sg1, *, dpad):
    wid = _wid()
    pk = (pk0_v, pk1_v)
    sb = (sb0_v, sb1_v)
    rows = (r0_v, r1_v)
    sp = (sp0, sp1)
    sg = (sg0, sg1)

    pltpu.sync_copy(cnt_hbm.at[wid], cnt_v)
    count = cnt_v[pl.ds(0, L)][0]
    nb = (count + G - 1) // G

    def init_body(i, _):
        agg_v[pl.ds(i * L, L)] = jnp.full((L,), NEG, jnp.float32)
        return 0
    lax.fori_loop(0, NPW * dpad // L, init_body, 0)

    def unpack(p):
        for k in range(G // L):
            sb[p][pl.ds(k * L, L)] = pk[p][pl.ds(k * L, L)] & 0xFFFF

    def start_load(b, p):
        pltpu.async_copy(part_hbm.at[wid, pl.ds(b * G, G)],
                         pk[p].at[pl.ds(0, G)], sp[p])

    def wait_load(b, p):
        pltpu.make_async_copy(part_hbm.at[wid, pl.ds(b * G, G)],
                              pk[p].at[pl.ds(0, G)], sp[p]).wait()

    def start_gather(p):
        pltpu.async_copy(hp_hbm.at[sb[p]], rows[p], sg[p])

    def wait_gather(p):
        pltpu.make_async_copy(hp_hbm.at[sb[p]], rows[p], sg[p]).wait()

    def drain(p, nd):
        def edge_body(j, _):
            pval = pk[p][pl.ds(j, L)][0]
            loc = pval >> 16
            rb = loc * dpad
            for dblk in range(dpad // L):
                off = dblk * L
                agg_v[pl.ds(rb + off, L)] = jnp.maximum(
                    agg_v[pl.ds(rb + off, L)], rows[p][j, pl.ds(off, L)])
            return 0
        lax.fori_loop(0, nd, edge_body, 0)

    @pl.when(nb > 0)
    def _prologue():
        pltpu.sync_copy(part_hbm.at[wid, pl.ds(0, G)], pk[0].at[pl.ds(0, G)])
        unpack(0)
        start_gather(0)

        @pl.when(nb > 1)
        def _():
            start_load(1, 1)

    def pair_body(i, _):
        for p in range(2):
            b = 2 * i + p
            q = 1 - p

            @pl.when(b < nb)
            def _do():
                @pl.when(b + 1 < nb)
                def _():
                    wait_load(b + 1, q)
                    unpack(q)
                    start_gather(q)

                wait_gather(p)
                drain(p, jnp.minimum(G, count - b * G))

                @pl.when(b + 2 < nb)
                def _():
                    start_load(b + 2, p)
        return 0

    lax.fori_loop(0, (nb + 1) // 2, pair_body, 0)
    pltpu.sync_copy(agg_v, out_hbm.at[pl.ds(wid * NPW * dpad, NPW * dpad)])


@functools.lru_cache(maxsize=None)
def _make_segmax(dpad):
    return pl.kernel(
        functools.partial(_segmax_body, dpad=dpad),
        out_type=jax.ShapeDtypeStruct((NPAD * dpad,), jnp.float32),
        mesh=_MESH,
        compiler_params=_SC_PARAMS,
        scratch_types=[
            pltpu.VMEM((NPW * dpad,), jnp.float32),
            pltpu.VMEM((G + L,), jnp.int32),
            pltpu.VMEM((G + L,), jnp.int32),
            pltpu.VMEM((G,), jnp.int32),
            pltpu.VMEM((G,), jnp.int32),
            pltpu.VMEM((G, dpad), jnp.float32),
            pltpu.VMEM((G, dpad), jnp.float32),
            pltpu.VMEM((L,), jnp.int32),
            pltpu.SemaphoreType.DMA,
            pltpu.SemaphoreType.DMA,
            pltpu.SemaphoreType.DMA,
            pltpu.SemaphoreType.DMA,
        ],
    )


def _pad2(w, r, c):
    return jnp.pad(w, ((0, r - w.shape[0]), (0, c - w.shape[1])))


def kernel(x, adj, params):
    dpi_l = [128, 128, 128, 128, 128]
    dpo_l = [128, 128, 128, 128, 128]

    h = _normalize(x)
    part, cnts = _partition(adj)

    for i in range(5):
        dpi, dpo = dpi_l[i], dpo_l[i]
        wp = _pad2(params['Wp%d' % i], dpi, dpi)
        bp = jnp.pad(params['bp%d' % i], (0, dpi - params['bp%d' % i].shape[0]))
        ws = _pad2(params['Ws%d' % i], dpi, dpo)
        wn = _pad2(params['Wn%d' % i], dpi, dpo)
        b = jnp.pad(params['b%d' % i], (0, dpo - params['b%d' % i].shape[0]))

        hp = _pool(h, wp, bp.reshape(1, dpi))
        agg_flat = _make_segmax(dpi)(hp, part, cnts)
        agg = agg_flat.reshape(NPAD, dpi)[:N]
        h = _combine(h, agg, ws, wn, b.reshape(1, dpo), relu=(i < 4))

    return h


# speedup vs baseline: 3.4364x; 1.3575x over previous
import functools

import jax
import jax.numpy as jnp
from jax import lax
from jax.experimental import pallas as pl
from jax.experimental.pallas import tpu as pltpu
from jax.experimental.pallas import tpu_sc as plsc

N = 10000
E = 320000
NC, NS, L = 2, 16, 16
NW = NC * NS
NPW = (N + NW - 1) // NW
NPAD = NW * NPW
G = 128
CAP = G + 2 * L
CHUNK = 6400
NEG = -3.0e38

BN = 1000


def _norm_body(x_ref, o_ref):
    x = x_ref[...]
    ss = jnp.sum(x * x, axis=1, keepdims=True)
    nrm = jnp.maximum(jnp.sqrt(ss), 1e-12)
    o_ref[...] = x / nrm


def _normalize(x):
    return pl.pallas_call(
        _norm_body,
        out_shape=jax.ShapeDtypeStruct((N, 128), jnp.float32),
        grid=(N // BN,),
        in_specs=[pl.BlockSpec((BN, 128), lambda i: (i, 0))],
        out_specs=pl.BlockSpec((BN, 128), lambda i: (i, 0)),
    )(x)


def _pool_body(h_ref, w_ref, b_ref, o_ref):
    acc = jnp.dot(h_ref[...], w_ref[...], preferred_element_type=jnp.float32)
    o_ref[...] = jnp.maximum(acc + b_ref[...], 0.0)


def _pool(h, w, b):
    dpi = h.shape[1]
    return pl.pallas_call(
        _pool_body,
        out_shape=jax.ShapeDtypeStruct((N, dpi), jnp.float32),
        grid=(N // BN,),
        in_specs=[
            pl.BlockSpec((BN, dpi), lambda i: (i, 0)),
            pl.BlockSpec((dpi, dpi), lambda i: (0, 0)),
            pl.BlockSpec((1, dpi), lambda i: (0, 0)),
        ],
        out_specs=pl.BlockSpec((BN, dpi), lambda i: (i, 0)),
    )(h, w, b)


def _combine_body(h_ref, a_ref, ws_ref, wn_ref, b_ref, o_ref, *, relu):
    a = a_ref[...]
    a = jnp.where(a < -1e30, 0.0, a)
    acc = jnp.dot(h_ref[...], ws_ref[...], preferred_element_type=jnp.float32)
    acc += jnp.dot(a, wn_ref[...], preferred_element_type=jnp.float32)
    acc += b_ref[...]
    if relu:
        acc = jnp.maximum(acc, 0.0)
    o_ref[...] = acc


def _combine(h, agg, ws, wn, b, relu):
    dpi = h.shape[1]
    dpo = ws.shape[1]
    return pl.pallas_call(
        functools.partial(_combine_body, relu=relu),
        out_shape=jax.ShapeDtypeStruct((N, dpo), jnp.float32),
        grid=(N // BN,),
        in_specs=[
            pl.BlockSpec((BN, dpi), lambda i: (i, 0)),
            pl.BlockSpec((BN, dpi), lambda i: (i, 0)),
            pl.BlockSpec((dpi, dpo), lambda i: (0, 0)),
            pl.BlockSpec((dpi, dpo), lambda i: (0, 0)),
            pl.BlockSpec((1, dpo), lambda i: (0, 0)),
        ],
        out_specs=pl.BlockSpec((BN, dpo), lambda i: (i, 0)),
    )(h, agg, ws, wn, b)


_MESH = plsc.VectorSubcoreMesh(
    core_axis_name="c", subcore_axis_name="s", num_cores=NC, num_subcores=NS)
_SC_PARAMS = pltpu.CompilerParams(needs_layout_passes=False)


def _wid():
    return lax.axis_index("s") * NC + lax.axis_index("c")


NCHUNK = E // CHUNK


def _partition_body(adj_hbm, part_hbm, cnt_hbm,
                    ad0_v, ad1_v, sel_p, cnt_v, sem0, sem1):
    wid = _wid()
    lo = wid * NPW
    hi = lo + NPW
    ad_v = (ad0_v, ad1_v)
    sems = (sem0, sem1)

    for i in range(CAP // L):
        sel_p[pl.ds(i * L, L)] = jnp.zeros((L,), jnp.int32)

    def grp_body(ad, g, carry):
        nsel, nfired = carry
        base = g * 2 * L
        s16a = ad[0, pl.ds(base, L)]
        d16a = ad[1, pl.ds(base, L)]
        s16b = ad[0, pl.ds(base + L, L)]
        d16b = ad[1, pl.ds(base + L, L)]
        ma = (d16a >= lo) & (d16a < hi)
        mb = (d16b >= lo) & (d16b < hi)
        cuma = plsc.cumsum(ma.astype(jnp.int32))
        cumb = plsc.cumsum(mb.astype(jnp.int32))
        packa = s16a | ((d16a - lo) << 16)
        packb = s16b | ((d16b - lo) << 16)
        na = cuma[L - 1]
        plsc.store_scatter(sel_p, [nsel + cuma - 1], packa, mask=ma)
        plsc.store_scatter(sel_p, [nsel + na + cumb - 1], packb, mask=mb)
        nsel = nsel + na + cumb[L - 1]

        full = nsel >= G

        @pl.when(full)
        def _fire():
            pltpu.sync_copy(sel_p.at[pl.ds(0, G)],
                            part_hbm.at[wid, pl.ds(nfired * G, G)])
            sel_p[pl.ds(0, L)] = sel_p[pl.ds(G, L)]
            sel_p[pl.ds(L, L)] = sel_p[pl.ds(G + L, L)]

        nsel = jnp.where(full, nsel - G, nsel)
        nfired = jnp.where(full, nfired + 1, nfired)
        return nsel, nfired

    pltpu.async_copy(adj_hbm.at[:, pl.ds(0, CHUNK)], ad0_v, sem0)
    pltpu.async_copy(adj_hbm.at[:, pl.ds(CHUNK, CHUNK)], ad1_v, sem1)

    def pair_body(i, carry):
        for p in range(2):
            b = 2 * i + p
            pltpu.make_async_copy(
                adj_hbm.at[:, pl.ds(b * CHUNK, CHUNK)], ad_v[p], sems[p]
            ).wait()
            carry = lax.fori_loop(
                0, CHUNK // (2 * L), functools.partial(grp_body, ad_v[p]),
                carry)

            @pl.when(b + 2 < NCHUNK)
            def _refill():
                pltpu.async_copy(
                    adj_hbm.at[:, pl.ds((b + 2) * CHUNK, CHUNK)],
                    ad_v[p], sems[p])
        return carry

    assert NCHUNK % 2 == 0
    nsel, nfired = lax.fori_loop(0, NCHUNK // 2, pair_body, (0, 0))

    @pl.when(nsel > 0)
    def _final():
        pltpu.sync_copy(sel_p.at[pl.ds(0, G)],
                        part_hbm.at[wid, pl.ds(nfired * G, G)])

    count = nfired * G + nsel
    cnt_v[pl.ds(0, L)] = jnp.full((L,), 1, jnp.int32) * count
    pltpu.sync_copy(cnt_v, cnt_hbm.at[wid])


_partition = pl.kernel(
    _partition_body,
    out_type=[
        jax.ShapeDtypeStruct((NW, E), jnp.int32),
        jax.ShapeDtypeStruct((NW, L), jnp.int32),
    ],
    mesh=_MESH,
    compiler_params=_SC_PARAMS,
    scratch_types=[
        pltpu.VMEM((2, CHUNK), jnp.int32),
        pltpu.VMEM((2, CHUNK), jnp.int32),
        pltpu.VMEM((CAP,), jnp.int32),
        pltpu.VMEM((L,), jnp.int32),
        pltpu.SemaphoreType.DMA,
        pltpu.SemaphoreType.DMA,
    ],
)


def _segmax_body(hp_hbm, part_hbm, cnt_hbm, out_hbm,
                 agg_v, pk0_v, pk1_v, sb0_v, sb1_v, r0_v, r1_v, cnt_v,
                 sp0, sp1, sg0, sg1, *, dpad):
    wid = _wid()
    pk = (pk0_v, pk1_v)
    sb = (sb0_v, sb1_v)
    rows = (r0_v, r1_v)
    sp = (sp0, sp1)
    sg = (sg0, sg1)

    pltpu.sync_copy(cnt_hbm.at[wid], cnt_v)
    count = cnt_v[pl.ds(0, L)][0]
    nb = (count + G - 1) // G

    def init_body(i, _):
        agg_v[pl.ds(i * L, L)] = jnp.full((L,), NEG, jnp.float32)
        return 0
    lax.fori_loop(0, NPW * dpad // L, init_body, 0)

    def unpack(p):
        for k in range(G // L):
            sb[p][pl.ds(k * L, L)] = pk[p][pl.ds(k * L, L)] & 0xFFFF

    def start_load(b, p):
        pltpu.async_copy(part_hbm.at[wid, pl.ds(b * G, G)],
                         pk[p].at[pl.ds(0, G)], sp[p])

    def wait_load(b, p):
        pltpu.make_async_copy(part_hbm.at[wid, pl.ds(b * G, G)],
                              pk[p].at[pl.ds(0, G)], sp[p]).wait()

    def start_gather(p):
        pltpu.async_copy(hp_hbm.at[sb[p]], rows[p], sg[p])

    def wait_gather(p):
        pltpu.make_async_copy(hp_hbm.at[sb[p]], rows[p], sg[p]).wait()

    def _upd(p, j, rb):
        for dblk in range(dpad // L):
            off = dblk * L
            agg_v[pl.ds(rb + off, L)] = jnp.maximum(
                agg_v[pl.ds(rb + off, L)], rows[p][j, pl.ds(off, L)])

    def drain(p, nd):
        @pl.when(nd == G)
        def _full():
            def blk_body(blk, _):
                jb = blk * L
                rbv = (pk[p][pl.ds(jb, L)] >> 16) * dpad
                for lane in range(L):
                    _upd(p, jb + lane, rbv[lane])
                return 0
            lax.fori_loop(0, G // L, blk_body, 0)

        @pl.when(nd < G)
        def _partial():
            def edge_body(j, _):
                pval = pk[p][pl.ds(j, L)][0]
                _upd(p, j, (pval >> 16) * dpad)
                return 0
            lax.fori_loop(0, nd, edge_body, 0)

    @pl.when(nb > 0)
    def _prologue():
        pltpu.sync_copy(part_hbm.at[wid, pl.ds(0, G)], pk[0].at[pl.ds(0, G)])
        unpack(0)
        start_gather(0)

        @pl.when(nb > 1)
        def _():
            start_load(1, 1)

    def pair_body(i, _):
        for p in range(2):
            b = 2 * i + p
            q = 1 - p

            @pl.when(b < nb)
            def _do():
                @pl.when(b + 1 < nb)
                def _():
                    wait_load(b + 1, q)
                    unpack(q)
                    start_gather(q)

                wait_gather(p)
                drain(p, jnp.minimum(G, count - b * G))

                @pl.when(b + 2 < nb)
                def _():
                    start_load(b + 2, p)
        return 0

    lax.fori_loop(0, (nb + 1) // 2, pair_body, 0)
    pltpu.sync_copy(agg_v, out_hbm.at[pl.ds(wid * NPW * dpad, NPW * dpad)])


@functools.lru_cache(maxsize=None)
def _make_segmax(dpad):
    return pl.kernel(
        functools.partial(_segmax_body, dpad=dpad),
        out_type=jax.ShapeDtypeStruct((NPAD * dpad,), jnp.float32),
        mesh=_MESH,
        compiler_params=_SC_PARAMS,
        scratch_types=[
            pltpu.VMEM((NPW * dpad,), jnp.float32),
            pltpu.VMEM((G + L,), jnp.int32),
            pltpu.VMEM((G + L,), jnp.int32),
            pltpu.VMEM((G,), jnp.int32),
            pltpu.VMEM((G,), jnp.int32),
            pltpu.VMEM((G, dpad), jnp.float32),
            pltpu.VMEM((G, dpad), jnp.float32),
            pltpu.VMEM((L,), jnp.int32),
            pltpu.SemaphoreType.DMA,
            pltpu.SemaphoreType.DMA,
            pltpu.SemaphoreType.DMA,
            pltpu.SemaphoreType.DMA,
        ],
    )


def _pad2(w, r, c):
    return jnp.pad(w, ((0, r - w.shape[0]), (0, c - w.shape[1])))


def kernel(x, adj, params):
    dpi_l = [128, 128, 128, 128, 128]
    dpo_l = [128, 128, 128, 128, 128]

    h = _normalize(x)
    part, cnts = _partition(adj)

    for i in range(5):
        dpi, dpo = dpi_l[i], dpo_l[i]
        wp = _pad2(params['Wp%d' % i], dpi, dpi)
        bp = jnp.pad(params['bp%d' % i], (0, dpi - params['bp%d' % i].shape[0]))
        ws = _pad2(params['Ws%d' % i], dpi, dpo)
        wn = _pad2(params['Wn%d' % i], dpi, dpo)
        b = jnp.pad(params['b%d' % i], (0, dpo - params['b%d' % i].shape[0]))

        hp = _pool(h, wp, bp.reshape(1, dpi))
        agg_flat = _make_segmax(dpi)(hp, part, cnts)
        agg = agg_flat.reshape(NPAD, dpi)[:N]
        h = _combine(h, agg, ws, wn, b.reshape(1, dpo), relu=(i < 4))

    return h


# speedup vs baseline: 5.8639x; 1.7064x over previous
import functools

import jax
import jax.numpy as jnp
from jax import lax
from jax.experimental import pallas as pl
from jax.experimental.pallas import tpu as pltpu
from jax.experimental.pallas import tpu_sc as plsc

N = 10000
E = 320000
NC, NS, L = 2, 16, 16
NW = NC * NS
NPW = (N + NW - 1) // NW
NPAD = NW * NPW
G = 128
CAP = G + 2 * L
CHUNK = 6400
NEG = -3.0e38

BN = 1000


def _norm_body(x_ref, o_ref):
    x = x_ref[...]
    ss = jnp.sum(x * x, axis=1, keepdims=True)
    nrm = jnp.maximum(jnp.sqrt(ss), 1e-12)
    o_ref[...] = x / nrm


def _normalize(x):
    return pl.pallas_call(
        _norm_body,
        out_shape=jax.ShapeDtypeStruct((N, 128), jnp.float32),
        grid=(N // BN,),
        in_specs=[pl.BlockSpec((BN, 128), lambda i: (i, 0))],
        out_specs=pl.BlockSpec((BN, 128), lambda i: (i, 0)),
    )(x)


def _pool_body(h_ref, w_ref, b_ref, o_ref):
    acc = jnp.dot(h_ref[...], w_ref[...], preferred_element_type=jnp.float32)
    o_ref[...] = jnp.maximum(acc + b_ref[...], 0.0)


def _pool(h, w, b):
    dpi = h.shape[1]
    return pl.pallas_call(
        _pool_body,
        out_shape=jax.ShapeDtypeStruct((N, dpi), jnp.float32),
        grid=(N // BN,),
        in_specs=[
            pl.BlockSpec((BN, dpi), lambda i: (i, 0)),
            pl.BlockSpec((dpi, dpi), lambda i: (0, 0)),
            pl.BlockSpec((1, dpi), lambda i: (0, 0)),
        ],
        out_specs=pl.BlockSpec((BN, dpi), lambda i: (i, 0)),
    )(h, w, b)


def _combine_body(h_ref, a_ref, ws_ref, wn_ref, b_ref, o_ref, *, relu):
    a = a_ref[...]
    a = jnp.where(a < -1e30, 0.0, a)
    acc = jnp.dot(h_ref[...], ws_ref[...], preferred_element_type=jnp.float32)
    acc += jnp.dot(a, wn_ref[...], preferred_element_type=jnp.float32)
    acc += b_ref[...]
    if relu:
        acc = jnp.maximum(acc, 0.0)
    o_ref[...] = acc


def _combine(h, agg, ws, wn, b, relu):
    dpi = h.shape[1]
    dpo = ws.shape[1]
    return pl.pallas_call(
        functools.partial(_combine_body, relu=relu),
        out_shape=jax.ShapeDtypeStruct((N, dpo), jnp.float32),
        grid=(N // BN,),
        in_specs=[
            pl.BlockSpec((BN, dpi), lambda i: (i, 0)),
            pl.BlockSpec((BN, dpi), lambda i: (i, 0)),
            pl.BlockSpec((dpi, dpo), lambda i: (0, 0)),
            pl.BlockSpec((dpi, dpo), lambda i: (0, 0)),
            pl.BlockSpec((1, dpo), lambda i: (0, 0)),
        ],
        out_specs=pl.BlockSpec((BN, dpo), lambda i: (i, 0)),
    )(h, agg, ws, wn, b)


_MESH = plsc.VectorSubcoreMesh(
    core_axis_name="c", subcore_axis_name="s", num_cores=NC, num_subcores=NS)
_SC_PARAMS = pltpu.CompilerParams(needs_layout_passes=False)


def _wid():
    return lax.axis_index("s") * NC + lax.axis_index("c")


NCHUNK = E // CHUNK


def _partition_body(adj_hbm, part_hbm, cnt_hbm,
                    ad0_v, ad1_v, sel_p, cnt_v, sem0, sem1):
    wid = _wid()
    lo = wid * NPW
    hi = lo + NPW
    ad_v = (ad0_v, ad1_v)
    sems = (sem0, sem1)

    for i in range(CAP // L):
        sel_p[pl.ds(i * L, L)] = jnp.zeros((L,), jnp.int32)

    def grp_body(ad, g, carry):
        nsel, nfired = carry
        base = g * 2 * L
        s16a = ad[0, pl.ds(base, L)]
        d16a = ad[1, pl.ds(base, L)]
        s16b = ad[0, pl.ds(base + L, L)]
        d16b = ad[1, pl.ds(base + L, L)]
        ma = (d16a >= lo) & (d16a < hi)
        mb = (d16b >= lo) & (d16b < hi)
        cuma = plsc.cumsum(ma.astype(jnp.int32))
        cumb = plsc.cumsum(mb.astype(jnp.int32))
        packa = s16a | ((d16a - lo) << 16)
        packb = s16b | ((d16b - lo) << 16)
        na = cuma[L - 1]
        plsc.store_scatter(sel_p, [nsel + cuma - 1], packa, mask=ma)
        plsc.store_scatter(sel_p, [nsel + na + cumb - 1], packb, mask=mb)
        nsel = nsel + na + cumb[L - 1]

        full = nsel >= G

        @pl.when(full)
        def _fire():
            pltpu.sync_copy(sel_p.at[pl.ds(0, G)],
                            part_hbm.at[wid, pl.ds(nfired * G, G)])
            sel_p[pl.ds(0, L)] = sel_p[pl.ds(G, L)]
            sel_p[pl.ds(L, L)] = sel_p[pl.ds(G + L, L)]

        nsel = jnp.where(full, nsel - G, nsel)
        nfired = jnp.where(full, nfired + 1, nfired)
        return nsel, nfired

    pltpu.async_copy(adj_hbm.at[:, pl.ds(0, CHUNK)], ad0_v, sem0)
    pltpu.async_copy(adj_hbm.at[:, pl.ds(CHUNK, CHUNK)], ad1_v, sem1)

    def pair_body(i, carry):
        for p in range(2):
            b = 2 * i + p
            pltpu.make_async_copy(
                adj_hbm.at[:, pl.ds(b * CHUNK, CHUNK)], ad_v[p], sems[p]
            ).wait()
            carry = lax.fori_loop(
                0, CHUNK // (2 * L), functools.partial(grp_body, ad_v[p]),
                carry)

            @pl.when(b + 2 < NCHUNK)
            def _refill():
                pltpu.async_copy(
                    adj_hbm.at[:, pl.ds((b + 2) * CHUNK, CHUNK)],
                    ad_v[p], sems[p])
        return carry

    assert NCHUNK % 2 == 0
    nsel, nfired = lax.fori_loop(0, NCHUNK // 2, pair_body, (0, 0))

    @pl.when(nsel > 0)
    def _final():
        pltpu.sync_copy(sel_p.at[pl.ds(0, G)],
                        part_hbm.at[wid, pl.ds(nfired * G, G)])

    count = nfired * G + nsel
    cnt_v[pl.ds(0, L)] = jnp.full((L,), 1, jnp.int32) * count
    pltpu.sync_copy(cnt_v, cnt_hbm.at[wid])


_partition = pl.kernel(
    _partition_body,
    out_type=[
        jax.ShapeDtypeStruct((NW, E), jnp.int32),
        jax.ShapeDtypeStruct((NW, L), jnp.int32),
    ],
    mesh=_MESH,
    compiler_params=_SC_PARAMS,
    scratch_types=[
        pltpu.VMEM((2, CHUNK), jnp.int32),
        pltpu.VMEM((2, CHUNK), jnp.int32),
        pltpu.VMEM((CAP,), jnp.int32),
        pltpu.VMEM((L,), jnp.int32),
        pltpu.SemaphoreType.DMA,
        pltpu.SemaphoreType.DMA,
    ],
)


def _segmax_body(hp_hbm, part_hbm, cnt_hbm, out_hbm,
                 agg_v, pk0_v, pk1_v, sb0_v, sb1_v, r0_v, r1_v, cnt_v,
                 sp0, sp1, sg0, sg1, *, dpad):
    wid = _wid()
    pk = (pk0_v, pk1_v)
    sb = (sb0_v, sb1_v)
    rows = (r0_v, r1_v)
    sp = (sp0, sp1)
    sg = (sg0, sg1)

    pltpu.sync_copy(cnt_hbm.at[wid], cnt_v)
    count = cnt_v[pl.ds(0, L)][0]
    nb = (count + G - 1) // G

    def init_body(i, _):
        agg_v[pl.ds(i * L, L)] = jnp.full((L,), NEG, jnp.float32)
        return 0
    lax.fori_loop(0, NPW * dpad // L, init_body, 0)

    def unpack(p):
        for k in range(G // L):
            sb[p][pl.ds(k * L, L)] = pk[p][pl.ds(k * L, L)] & 0xFFFF

    def start_load(b, p):
        pltpu.async_copy(part_hbm.at[wid, pl.ds(b * G, G)],
                         pk[p].at[pl.ds(0, G)], sp[p])

    def wait_load(b, p):
        pltpu.make_async_copy(part_hbm.at[wid, pl.ds(b * G, G)],
                              pk[p].at[pl.ds(0, G)], sp[p]).wait()

    def start_gather(p):
        pltpu.async_copy(hp_hbm.at[sb[p]], rows[p], sg[p])

    def wait_gather(p):
        pltpu.make_async_copy(hp_hbm.at[sb[p]], rows[p], sg[p]).wait()

    def _upd(p, j, rb):
        nblk = dpad // L
        rv = [rows[p][j, pl.ds(k * L, L)] for k in range(nblk)]
        av = [agg_v[pl.ds(rb + k * L, L)] for k in range(nblk)]
        for k in range(nblk):
            agg_v[pl.ds(rb + k * L, L)] = jnp.maximum(av[k], rv[k])

    def drain(p, nd):
        @pl.when(nd == G)
        def _full():
            def blk_body(blk, _):
                jb = blk * L
                rbv = (pk[p][pl.ds(jb, L)] >> 16) * dpad
                for lane in range(L):
                    _upd(p, jb + lane, rbv[lane])
                return 0
            lax.fori_loop(0, G // L, blk_body, 0)

        @pl.when(nd < G)
        def _partial():
            def edge_body(j, _):
                pval = pk[p][pl.ds(j, L)][0]
                _upd(p, j, (pval >> 16) * dpad)
                return 0
            lax.fori_loop(0, nd, edge_body, 0)

    @pl.when(nb > 0)
    def _prologue():
        pltpu.sync_copy(part_hbm.at[wid, pl.ds(0, G)], pk[0].at[pl.ds(0, G)])
        unpack(0)
        start_gather(0)

        @pl.when(nb > 1)
        def _():
            start_load(1, 1)

    def pair_body(i, _):
        for p in range(2):
            b = 2 * i + p
            q = 1 - p

            @pl.when(b < nb)
            def _do():
                @pl.when(b + 1 < nb)
                def _():
                    wait_load(b + 1, q)
                    unpack(q)
                    start_gather(q)

                wait_gather(p)
                drain(p, jnp.minimum(G, count - b * G))

                @pl.when(b + 2 < nb)
                def _():
                    start_load(b + 2, p)
        return 0

    lax.fori_loop(0, (nb + 1) // 2, pair_body, 0)
    pltpu.sync_copy(agg_v, out_hbm.at[pl.ds(wid * NPW * dpad, NPW * dpad)])


@functools.lru_cache(maxsize=None)
def _make_segmax(dpad):
    return pl.kernel(
        functools.partial(_segmax_body, dpad=dpad),
        out_type=jax.ShapeDtypeStruct((NPAD * dpad,), jnp.float32),
        mesh=_MESH,
        compiler_params=_SC_PARAMS,
        scratch_types=[
            pltpu.VMEM((NPW * dpad,), jnp.float32),
            pltpu.VMEM((G + L,), jnp.int32),
            pltpu.VMEM((G + L,), jnp.int32),
            pltpu.VMEM((G,), jnp.int32),
            pltpu.VMEM((G,), jnp.int32),
            pltpu.VMEM((G, dpad), jnp.float32),
            pltpu.VMEM((G, dpad), jnp.float32),
            pltpu.VMEM((L,), jnp.int32),
            pltpu.SemaphoreType.DMA,
            pltpu.SemaphoreType.DMA,
            pltpu.SemaphoreType.DMA,
            pltpu.SemaphoreType.DMA,
        ],
    )


def _pad2(w, r, c):
    return jnp.pad(w, ((0, r - w.shape[0]), (0, c - w.shape[1])))


def kernel(x, adj, params):
    dpi_l = [128, 128, 128, 128, 128]
    dpo_l = [128, 128, 128, 128, 128]

    h = _normalize(x)
    part, cnts = _partition(adj)

    for i in range(5):
        dpi, dpo = dpi_l[i], dpo_l[i]
        wp = _pad2(params['Wp%d' % i], dpi, dpi)
        bp = jnp.pad(params['bp%d' % i], (0, dpi - params['bp%d' % i].shape[0]))
        ws = _pad2(params['Ws%d' % i], dpi, dpo)
        wn = _pad2(params['Wn%d' % i], dpi, dpo)
        b = jnp.pad(params['b%d' % i], (0, dpo - params['b%d' % i].shape[0]))

        hp = _pool(h, wp, bp.reshape(1, dpi))
        agg_flat = _make_segmax(dpi)(hp, part, cnts)
        agg = agg_flat.reshape(NPAD, dpi)[:N]
        h = _combine(h, agg, ws, wn, b.reshape(1, dpo), relu=(i < 4))

    return h


# speedup vs baseline: 6.6467x; 1.1335x over previous
import functools

import jax
import jax.numpy as jnp
from jax import lax
from jax.experimental import pallas as pl
from jax.experimental.pallas import tpu as pltpu
from jax.experimental.pallas import tpu_sc as plsc

N = 10000
E = 320000
NC, NS, L = 2, 16, 16
NW = NC * NS
NPW = 320
NPAD = NW * NPW
G = 128
CAP = G + 2 * L
CHUNK = 6400
NEG = -3.0e38

BN = 1000


def _norm_body(x_ref, o_ref):
    x = x_ref[...]
    ss = jnp.sum(x * x, axis=1, keepdims=True)
    nrm = jnp.maximum(jnp.sqrt(ss), 1e-12)
    o_ref[...] = x / nrm


def _normalize(x):
    return pl.pallas_call(
        _norm_body,
        out_shape=jax.ShapeDtypeStruct((N, 128), jnp.float32),
        grid=(N // BN,),
        in_specs=[pl.BlockSpec((BN, 128), lambda i: (i, 0))],
        out_specs=pl.BlockSpec((BN, 128), lambda i: (i, 0)),
    )(x)


def _pool_body(h_ref, w_ref, b_ref, o_ref):
    acc = jnp.dot(h_ref[...], w_ref[...], preferred_element_type=jnp.float32)
    o_ref[...] = jnp.maximum(acc + b_ref[...], 0.0)


def _pool(h, w, b):
    dpi, dpo = w.shape
    return pl.pallas_call(
        _pool_body,
        out_shape=jax.ShapeDtypeStruct((N, dpo), jnp.float32),
        grid=(N // BN,),
        in_specs=[
            pl.BlockSpec((BN, dpi), lambda i: (i, 0)),
            pl.BlockSpec((dpi, dpo), lambda i: (0, 0)),
            pl.BlockSpec((1, dpo), lambda i: (0, 0)),
        ],
        out_specs=pl.BlockSpec((BN, dpo), lambda i: (i, 0)),
    )(h, w, b)


def _combine_body(h_ref, a_ref, ws_ref, wn_ref, b_ref, o_ref, *, relu):
    a = a_ref[...]
    a = jnp.where(a < -1e30, 0.0, a)
    acc = jnp.dot(h_ref[...], ws_ref[...], preferred_element_type=jnp.float32)
    acc += jnp.dot(a, wn_ref[...], preferred_element_type=jnp.float32)
    acc += b_ref[...]
    if relu:
        acc = jnp.maximum(acc, 0.0)
    o_ref[...] = acc


def _combine(h, agg, ws, wn, b, relu):
    dpi = h.shape[1]
    dact = agg.shape[1]
    dpo = ws.shape[1]
    return pl.pallas_call(
        functools.partial(_combine_body, relu=relu),
        out_shape=jax.ShapeDtypeStruct((N, dpo), jnp.float32),
        grid=(N // BN,),
        in_specs=[
            pl.BlockSpec((BN, dpi), lambda i: (i, 0)),
            pl.BlockSpec((BN, dact), lambda i: (i, 0)),
            pl.BlockSpec((dpi, dpo), lambda i: (0, 0)),
            pl.BlockSpec((dact, dpo), lambda i: (0, 0)),
            pl.BlockSpec((1, dpo), lambda i: (0, 0)),
        ],
        out_specs=pl.BlockSpec((BN, dpo), lambda i: (i, 0)),
    )(h, agg, ws, wn, b)


_MESH = plsc.VectorSubcoreMesh(
    core_axis_name="c", subcore_axis_name="s", num_cores=NC, num_subcores=NS)
_SC_PARAMS = pltpu.CompilerParams(needs_layout_passes=False)


def _wid():
    return lax.axis_index("s") * NC + lax.axis_index("c")


NCHUNK = E // CHUNK


def _partition_body(adj_hbm, part_hbm, cnt_hbm,
                    ad0_v, ad1_v, sel_p, cnt_v, sem0, sem1):
    wid = _wid()
    lo = wid * NPW
    hi = lo + NPW
    ad_v = (ad0_v, ad1_v)
    sems = (sem0, sem1)

    for i in range(CAP // L):
        sel_p[pl.ds(i * L, L)] = jnp.zeros((L,), jnp.int32)

    def grp_body(ad, g, carry):
        nsel, nfired = carry
        base = g * 2 * L
        s16a = ad[0, pl.ds(base, L)]
        d16a = ad[1, pl.ds(base, L)]
        s16b = ad[0, pl.ds(base + L, L)]
        d16b = ad[1, pl.ds(base + L, L)]
        ma = (d16a >= lo) & (d16a < hi)
        mb = (d16b >= lo) & (d16b < hi)
        cuma = plsc.cumsum(ma.astype(jnp.int32))
        cumb = plsc.cumsum(mb.astype(jnp.int32))
        packa = s16a | ((d16a - lo) << 16)
        packb = s16b | ((d16b - lo) << 16)
        na = cuma[L - 1]
        plsc.store_scatter(sel_p, [nsel + cuma - 1], packa, mask=ma)
        plsc.store_scatter(sel_p, [nsel + na + cumb - 1], packb, mask=mb)
        nsel = nsel + na + cumb[L - 1]

        full = nsel >= G

        @pl.when(full)
        def _fire():
            pltpu.sync_copy(sel_p.at[pl.ds(0, G)],
                            part_hbm.at[wid, pl.ds(nfired * G, G)])
            sel_p[pl.ds(0, L)] = sel_p[pl.ds(G, L)]
            sel_p[pl.ds(L, L)] = sel_p[pl.ds(G + L, L)]

        nsel = jnp.where(full, nsel - G, nsel)
        nfired = jnp.where(full, nfired + 1, nfired)
        return nsel, nfired

    pltpu.async_copy(adj_hbm.at[:, pl.ds(0, CHUNK)], ad0_v, sem0)
    pltpu.async_copy(adj_hbm.at[:, pl.ds(CHUNK, CHUNK)], ad1_v, sem1)

    def pair_body(i, carry):
        for p in range(2):
            b = 2 * i + p
            pltpu.make_async_copy(
                adj_hbm.at[:, pl.ds(b * CHUNK, CHUNK)], ad_v[p], sems[p]
            ).wait()
            carry = lax.fori_loop(
                0, CHUNK // (2 * L), functools.partial(grp_body, ad_v[p]),
                carry)

            @pl.when(b + 2 < NCHUNK)
            def _refill():
                pltpu.async_copy(
                    adj_hbm.at[:, pl.ds((b + 2) * CHUNK, CHUNK)],
                    ad_v[p], sems[p])
        return carry

    assert NCHUNK % 2 == 0
    nsel, nfired = lax.fori_loop(0, NCHUNK // 2, pair_body, (0, 0))

    @pl.when(nsel > 0)
    def _final():
        pltpu.sync_copy(sel_p.at[pl.ds(0, G)],
                        part_hbm.at[wid, pl.ds(nfired * G, G)])

    count = nfired * G + nsel
    cnt_v[pl.ds(0, L)] = jnp.full((L,), 1, jnp.int32) * count
    pltpu.sync_copy(cnt_v, cnt_hbm.at[wid])


_partition = pl.kernel(
    _partition_body,
    out_type=[
        jax.ShapeDtypeStruct((NW, E), jnp.int32),
        jax.ShapeDtypeStruct((NW, L), jnp.int32),
    ],
    mesh=_MESH,
    compiler_params=_SC_PARAMS,
    scratch_types=[
        pltpu.VMEM((2, CHUNK), jnp.int32),
        pltpu.VMEM((2, CHUNK), jnp.int32),
        pltpu.VMEM((CAP,), jnp.int32),
        pltpu.VMEM((L,), jnp.int32),
        pltpu.SemaphoreType.DMA,
        pltpu.SemaphoreType.DMA,
    ],
)


def _segmax_body(hp_hbm, part_hbm, cnt_hbm, out_hbm,
                 agg_v, pk0_v, pk1_v, sb0_v, sb1_v, r0_v, r1_v, cnt_v,
                 sp0, sp1, sg0, sg1, *, dact):
    wid = _wid()
    pk = (pk0_v, pk1_v)
    sb = (sb0_v, sb1_v)
    rows = (r0_v, r1_v)
    sp = (sp0, sp1)
    sg = (sg0, sg1)

    pltpu.sync_copy(cnt_hbm.at[wid], cnt_v)
    count = cnt_v[pl.ds(0, L)][0]
    nb = (count + G - 1) // G

    neg = jnp.full((L,), NEG, jnp.float32)

    def init_body(r, _):
        for k in range(dact // L):
            agg_v[r, pl.ds(k * L, L)] = neg
        return 0
    lax.fori_loop(0, NPW, init_body, 0)

    def unpack(p):
        for k in range(G // L):
            sb[p][pl.ds(k * L, L)] = pk[p][pl.ds(k * L, L)] & 0xFFFF

    def start_load(b, p):
        pltpu.async_copy(part_hbm.at[wid, pl.ds(b * G, G)],
                         pk[p].at[pl.ds(0, G)], sp[p])

    def wait_load(b, p):
        pltpu.make_async_copy(part_hbm.at[wid, pl.ds(b * G, G)],
                              pk[p].at[pl.ds(0, G)], sp[p]).wait()

    def start_gather(p):
        pltpu.async_copy(hp_hbm.at[sb[p]], rows[p], sg[p])

    def wait_gather(p):
        pltpu.make_async_copy(hp_hbm.at[sb[p]], rows[p], sg[p]).wait()

    def _upd(p, j, loc):
        nblk = dact // L
        rv = [rows[p][j, pl.ds(k * L, L)] for k in range(nblk)]
        av = [agg_v[loc, pl.ds(k * L, L)] for k in range(nblk)]
        for k in range(nblk):
            agg_v[loc, pl.ds(k * L, L)] = jnp.maximum(av[k], rv[k])

    def drain(p, nd):
        @pl.when(nd == G)
        def _full():
            def blk_body(blk, _):
                jb = blk * L
                locv = pk[p][pl.ds(jb, L)] >> 16
                for lane in range(L):
                    _upd(p, jb + lane, locv[lane])
                return 0
            lax.fori_loop(0, G // L, blk_body, 0)

        @pl.when(nd < G)
        def _partial():
            def edge_body(j, _):
                pval = pk[p][pl.ds(j, L)][0]
                _upd(p, j, pval >> 16)
                return 0
            lax.fori_loop(0, nd, edge_body, 0)

    @pl.when(nb > 0)
    def _prologue():
        pltpu.sync_copy(part_hbm.at[wid, pl.ds(0, G)], pk[0].at[pl.ds(0, G)])
        unpack(0)
        start_gather(0)

        @pl.when(nb > 1)
        def _():
            start_load(1, 1)

    def pair_body(i, _):
        for p in range(2):
            b = 2 * i + p
            q = 1 - p

            @pl.when(b < nb)
            def _do():
                @pl.when(b + 1 < nb)
                def _():
                    wait_load(b + 1, q)
                    unpack(q)
                    start_gather(q)

                wait_gather(p)
                drain(p, jnp.minimum(G, count - b * G))

                @pl.when(b + 2 < nb)
                def _():
                    start_load(b + 2, p)
        return 0

    lax.fori_loop(0, (nb + 1) // 2, pair_body, 0)
    pltpu.sync_copy(agg_v, out_hbm.at[pl.ds(wid * NPW, NPW)])


@functools.lru_cache(maxsize=None)
def _make_segmax(dact):
    return pl.kernel(
        functools.partial(_segmax_body, dact=dact),
        out_type=jax.ShapeDtypeStruct((NPAD, dact), jnp.float32),
        mesh=_MESH,
        compiler_params=_SC_PARAMS,
        scratch_types=[
            pltpu.VMEM((NPW, dact), jnp.float32),
            pltpu.VMEM((G + L,), jnp.int32),
            pltpu.VMEM((G + L,), jnp.int32),
            pltpu.VMEM((G,), jnp.int32),
            pltpu.VMEM((G,), jnp.int32),
            pltpu.VMEM((G, 128), jnp.float32),
            pltpu.VMEM((G, 128), jnp.float32),
            pltpu.VMEM((L,), jnp.int32),
            pltpu.SemaphoreType.DMA,
            pltpu.SemaphoreType.DMA,
            pltpu.SemaphoreType.DMA,
            pltpu.SemaphoreType.DMA,
        ],
    )


def _pad2(w, r, c):
    return jnp.pad(w, ((0, r - w.shape[0]), (0, c - w.shape[1])))


def kernel(x, adj, params):
    dpi_l = [128, 80, 80, 80, 80]
    dact_l = [128, 80, 80, 80, 80]
    dpo_l = [80, 80, 80, 80, 128]

    h = _normalize(x)
    part, cnts = _partition(adj)

    for i in range(5):
        dpi, dact, dpo = dpi_l[i], dact_l[i], dpo_l[i]
        wp = _pad2(params['Wp%d' % i], dpi, 128)
        bp = jnp.pad(params['bp%d' % i],
                     (0, 128 - params['bp%d' % i].shape[0]))
        ws = _pad2(params['Ws%d' % i], dpi, dpo)
        wn = _pad2(params['Wn%d' % i], dact, dpo)
        b = jnp.pad(params['b%d' % i], (0, dpo - params['b%d' % i].shape[0]))

        hp = _pool(h, wp, bp.reshape(1, 128))
        agg = _make_segmax(dact)(hp, part, cnts)[:N]
        h = _combine(h, agg, ws, wn, b.reshape(1, dpo), relu=(i < 4))

    return h


# speedup vs baseline: 7.6866x; 1.1565x over previous
import functools

import jax
import jax.numpy as jnp
from jax import lax
from jax.experimental import pallas as pl
from jax.experimental.pallas import tpu as pltpu
from jax.experimental.pallas import tpu_sc as plsc

N = 10000
E = 320000
NC, NS, L = 2, 16, 16
NW = NC * NS
NPW = 320
NPAD = NW * NPW
G = 128
FIRE = 3 * G
CAP = FIRE + 4 * L
CHUNK = 6400
NEG = -3.0e38

BN = 1000


def _norm_body(x_ref, o_ref):
    x = x_ref[...]
    ss = jnp.sum(x * x, axis=1, keepdims=True)
    nrm = jnp.maximum(jnp.sqrt(ss), 1e-12)
    o_ref[...] = x / nrm


def _normalize(x):
    return pl.pallas_call(
        _norm_body,
        out_shape=jax.ShapeDtypeStruct((N, 128), jnp.float32),
        grid=(N // BN,),
        in_specs=[pl.BlockSpec((BN, 128), lambda i: (i, 0))],
        out_specs=pl.BlockSpec((BN, 128), lambda i: (i, 0)),
    )(x)


def _pool_body(h_ref, w_ref, b_ref, o_ref):
    acc = jnp.dot(h_ref[...], w_ref[...], preferred_element_type=jnp.float32)
    o_ref[...] = jnp.maximum(acc + b_ref[...], 0.0)


def _pool(h, w, b):
    dpi, dpo = w.shape
    return pl.pallas_call(
        _pool_body,
        out_shape=jax.ShapeDtypeStruct((N, dpo), jnp.float32),
        grid=(N // BN,),
        in_specs=[
            pl.BlockSpec((BN, dpi), lambda i: (i, 0)),
            pl.BlockSpec((dpi, dpo), lambda i: (0, 0)),
            pl.BlockSpec((1, dpo), lambda i: (0, 0)),
        ],
        out_specs=pl.BlockSpec((BN, dpo), lambda i: (i, 0)),
    )(h, w, b)


def _combine_body(h_ref, a_ref, ws_ref, wn_ref, b_ref, o_ref, *, relu):
    a = a_ref[...]
    a = jnp.where(a < -1e30, 0.0, a)
    acc = jnp.dot(h_ref[...], ws_ref[...], preferred_element_type=jnp.float32)
    acc += jnp.dot(a, wn_ref[...], preferred_element_type=jnp.float32)
    acc += b_ref[...]
    if relu:
        acc = jnp.maximum(acc, 0.0)
    o_ref[...] = acc


def _combine(h, agg, ws, wn, b, relu):
    dpi = h.shape[1]
    dact = agg.shape[1]
    dpo = ws.shape[1]
    return pl.pallas_call(
        functools.partial(_combine_body, relu=relu),
        out_shape=jax.ShapeDtypeStruct((N, dpo), jnp.float32),
        grid=(N // BN,),
        in_specs=[
            pl.BlockSpec((BN, dpi), lambda i: (i, 0)),
            pl.BlockSpec((BN, dact), lambda i: (i, 0)),
            pl.BlockSpec((dpi, dpo), lambda i: (0, 0)),
            pl.BlockSpec((dact, dpo), lambda i: (0, 0)),
            pl.BlockSpec((1, dpo), lambda i: (0, 0)),
        ],
        out_specs=pl.BlockSpec((BN, dpo), lambda i: (i, 0)),
    )(h, agg, ws, wn, b)


_MESH = plsc.VectorSubcoreMesh(
    core_axis_name="c", subcore_axis_name="s", num_cores=NC, num_subcores=NS)
_SC_PARAMS = pltpu.CompilerParams(needs_layout_passes=False)


def _wid():
    return lax.axis_index("s") * NC + lax.axis_index("c")


NCHUNK = E // CHUNK


def _partition_body(adj_hbm, part_hbm, cnt_hbm,
                    ad0_v, ad1_v, sel_p, cnt_v, sem0, sem1):
    wid = _wid()
    lo = wid * NPW
    hi = lo + NPW
    ad_v = (ad0_v, ad1_v)
    sems = (sem0, sem1)

    for i in range(CAP // L):
        sel_p[pl.ds(i * L, L)] = jnp.zeros((L,), jnp.int32)

    def grp_body(ad, g, carry):
        nselv, nsel, nfired = carry
        base = g * 4 * L
        s4 = [ad[0, pl.ds(base + k * L, L)] for k in range(4)]
        d4 = [ad[1, pl.ds(base + k * L, L)] for k in range(4)]
        m4 = [(d >= lo) & (d < hi) for d in d4]
        cum4 = [plsc.cumsum(m.astype(jnp.int32)) for m in m4]
        pc4 = [plsc.all_reduce_population_count(m) for m in m4]
        pk4 = [s | ((d - lo) << 16) for s, d in zip(s4, d4)]
        off = nselv - 1
        for k in range(4):
            plsc.store_scatter(sel_p, [off + cum4[k]], pk4[k], mask=m4[k])
            off = off + pc4[k]
        tot = pc4[0] + pc4[1] + pc4[2] + pc4[3]
        nselv = nselv + tot
        nsel = nsel + tot[0]

        full = nsel >= FIRE

        @pl.when(full)
        def _fire():
            pltpu.sync_copy(sel_p.at[pl.ds(0, FIRE)],
                            part_hbm.at[wid, pl.ds(nfired * FIRE, FIRE)])
            for k in range(4):
                sel_p[pl.ds(k * L, L)] = sel_p[pl.ds(FIRE + k * L, L)]

        nselv = jnp.where(full, nselv - FIRE, nselv)
        nsel = jnp.where(full, nsel - FIRE, nsel)
        nfired = jnp.where(full, nfired + 1, nfired)
        return nselv, nsel, nfired

    pltpu.async_copy(adj_hbm.at[:, pl.ds(0, CHUNK)], ad0_v, sem0)
    pltpu.async_copy(adj_hbm.at[:, pl.ds(CHUNK, CHUNK)], ad1_v, sem1)

    def pair_body(i, carry):
        for p in range(2):
            b = 2 * i + p
            pltpu.make_async_copy(
                adj_hbm.at[:, pl.ds(b * CHUNK, CHUNK)], ad_v[p], sems[p]
            ).wait()
            carry = lax.fori_loop(
                0, CHUNK // (4 * L), functools.partial(grp_body, ad_v[p]),
                carry)

            @pl.when(b + 2 < NCHUNK)
            def _refill():
                pltpu.async_copy(
                    adj_hbm.at[:, pl.ds((b + 2) * CHUNK, CHUNK)],
                    ad_v[p], sems[p])
        return carry

    assert NCHUNK % 2 == 0
    _, nsel, nfired = lax.fori_loop(
        0, NCHUNK // 2, pair_body, (jnp.zeros((L,), jnp.int32), 0, 0))

    for j in range(FIRE // G):
        @pl.when(nsel > j * G)
        def _final(j=j):
            pltpu.sync_copy(
                sel_p.at[pl.ds(j * G, G)],
                part_hbm.at[wid, pl.ds(nfired * FIRE + j * G, G)])

    count = nfired * FIRE + nsel
    cnt_v[pl.ds(0, L)] = jnp.full((L,), 1, jnp.int32) * count
    pltpu.sync_copy(cnt_v, cnt_hbm.at[wid])


_partition = pl.kernel(
    _partition_body,
    out_type=[
        jax.ShapeDtypeStruct((NW, E), jnp.int32),
        jax.ShapeDtypeStruct((NW, L), jnp.int32),
    ],
    mesh=_MESH,
    compiler_params=_SC_PARAMS,
    scratch_types=[
        pltpu.VMEM((2, CHUNK), jnp.int32),
        pltpu.VMEM((2, CHUNK), jnp.int32),
        pltpu.VMEM((CAP,), jnp.int32),
        pltpu.VMEM((L,), jnp.int32),
        pltpu.SemaphoreType.DMA,
        pltpu.SemaphoreType.DMA,
    ],
)


def _segmax_body(hp_hbm, part_hbm, cnt_hbm, out_hbm,
                 agg_v, pk0_v, pk1_v, sb0_v, sb1_v, r0_v, r1_v, cnt_v,
                 sp0, sp1, sg0, sg1, *, dact):
    wid = _wid()
    pk = (pk0_v, pk1_v)
    sb = (sb0_v, sb1_v)
    rows = (r0_v, r1_v)
    sp = (sp0, sp1)
    sg = (sg0, sg1)

    pltpu.sync_copy(cnt_hbm.at[wid], cnt_v)
    count = cnt_v[pl.ds(0, L)][0]
    nb = (count + G - 1) // G

    neg = jnp.full((L,), NEG, jnp.float32)

    def init_body(r, _):
        for k in range(dact // L):
            agg_v[r, pl.ds(k * L, L)] = neg
        return 0
    lax.fori_loop(0, NPW, init_body, 0)

    def unpack(p):
        for k in range(G // L):
            sb[p][pl.ds(k * L, L)] = pk[p][pl.ds(k * L, L)] & 0xFFFF

    def start_load(b, p):
        pltpu.async_copy(part_hbm.at[wid, pl.ds(b * G, G)],
                         pk[p].at[pl.ds(0, G)], sp[p])

    def wait_load(b, p):
        pltpu.make_async_copy(part_hbm.at[wid, pl.ds(b * G, G)],
                              pk[p].at[pl.ds(0, G)], sp[p]).wait()

    def start_gather(p):
        pltpu.async_copy(hp_hbm.at[sb[p]], rows[p], sg[p])

    def wait_gather(p):
        pltpu.make_async_copy(hp_hbm.at[sb[p]], rows[p], sg[p]).wait()

    def _upd(p, j, loc):
        nblk = dact // L
        rv = [rows[p][j, pl.ds(k * L, L)] for k in range(nblk)]
        av = [agg_v[loc, pl.ds(k * L, L)] for k in range(nblk)]
        for k in range(nblk):
            agg_v[loc, pl.ds(k * L, L)] = jnp.maximum(av[k], rv[k])

    def drain(p, nd):
        @pl.when(nd == G)
        def _full():
            def blk_body(blk, _):
                jb = blk * L
                locv = pk[p][pl.ds(jb, L)] >> 16
                for lane in range(L):
                    _upd(p, jb + lane, locv[lane])
                return 0
            lax.fori_loop(0, G // L, blk_body, 0)

        @pl.when(nd < G)
        def _partial():
            def edge_body(j, _):
                pval = pk[p][pl.ds(j, L)][0]
                _upd(p, j, pval >> 16)
                return 0
            lax.fori_loop(0, nd, edge_body, 0)

    @pl.when(nb > 0)
    def _prologue():
        pltpu.sync_copy(part_hbm.at[wid, pl.ds(0, G)], pk[0].at[pl.ds(0, G)])
        unpack(0)
        start_gather(0)

        @pl.when(nb > 1)
        def _():
            start_load(1, 1)

    def pair_body(i, _):
        for p in range(2):
            b = 2 * i + p
            q = 1 - p

            @pl.when(b < nb)
            def _do():
                @pl.when(b + 1 < nb)
                def _():
                    wait_load(b + 1, q)
                    unpack(q)
                    start_gather(q)

                wait_gather(p)
                drain(p, jnp.minimum(G, count - b * G))

                @pl.when(b + 2 < nb)
                def _():
                    start_load(b + 2, p)
        return 0

    lax.fori_loop(0, (nb + 1) // 2, pair_body, 0)
    pltpu.sync_copy(agg_v, out_hbm.at[pl.ds(wid * NPW, NPW)])


@functools.lru_cache(maxsize=None)
def _make_segmax(dact):
    return pl.kernel(
        functools.partial(_segmax_body, dact=dact),
        out_type=jax.ShapeDtypeStruct((NPAD, dact), jnp.float32),
        mesh=_MESH,
        compiler_params=_SC_PARAMS,
        scratch_types=[
            pltpu.VMEM((NPW, dact), jnp.float32),
            pltpu.VMEM((G + L,), jnp.int32),
            pltpu.VMEM((G + L,), jnp.int32),
            pltpu.VMEM((G,), jnp.int32),
            pltpu.VMEM((G,), jnp.int32),
            pltpu.VMEM((G, 128), jnp.float32),
            pltpu.VMEM((G, 128), jnp.float32),
            pltpu.VMEM((L,), jnp.int32),
            pltpu.SemaphoreType.DMA,
            pltpu.SemaphoreType.DMA,
            pltpu.SemaphoreType.DMA,
            pltpu.SemaphoreType.DMA,
        ],
    )


def _pad2(w, r, c):
    return jnp.pad(w, ((0, r - w.shape[0]), (0, c - w.shape[1])))


def kernel(x, adj, params):
    dpi_l = [128, 80, 80, 80, 80]
    dact_l = [128, 80, 80, 80, 80]
    dpo_l = [80, 80, 80, 80, 128]

    h = _normalize(x)
    part, cnts = _partition(adj)

    for i in range(5):
        dpi, dact, dpo = dpi_l[i], dact_l[i], dpo_l[i]
        wp = _pad2(params['Wp%d' % i], dpi, 128)
        bp = jnp.pad(params['bp%d' % i],
                     (0, 128 - params['bp%d' % i].shape[0]))
        ws = _pad2(params['Ws%d' % i], dpi, dpo)
        wn = _pad2(params['Wn%d' % i], dact, dpo)
        b = jnp.pad(params['b%d' % i], (0, dpo - params['b%d' % i].shape[0]))

        hp = _pool(h, wp, bp.reshape(1, 128))
        agg = _make_segmax(dact)(hp, part, cnts)[:N]
        h = _combine(h, agg, ws, wn, b.reshape(1, dpo), relu=(i < 4))

    return h


# speedup vs baseline: 7.9015x; 1.0280x over previous
import functools

import jax
import jax.numpy as jnp
from jax import lax
from jax.experimental import pallas as pl
from jax.experimental.pallas import tpu as pltpu
from jax.experimental.pallas import tpu_sc as plsc

N = 10000
E = 320000
NC, NS, L = 2, 16, 16
NW = NC * NS
NPW = 320
NPAD = NW * NPW
G = 128
FIRE = 3 * G
CAP = FIRE + 4 * L
CHUNK = 6400
NEG = -3.0e38

BN = 1000


def _norm_pool_body(x_ref, wp_ref, bp_ref, h_ref, hp_ref):
    x = x_ref[...]
    ss = jnp.sum(x * x, axis=1, keepdims=True)
    nrm = jnp.maximum(jnp.sqrt(ss), 1e-12)
    h = x / nrm
    h_ref[...] = h
    acc = jnp.dot(h, wp_ref[...], preferred_element_type=jnp.float32)
    hp_ref[...] = jnp.maximum(acc + bp_ref[...], 0.0)


def _norm_pool(x, wp, bp):
    return pl.pallas_call(
        _norm_pool_body,
        out_shape=[jax.ShapeDtypeStruct((N, 128), jnp.float32),
                   jax.ShapeDtypeStruct((N, 128), jnp.float32)],
        grid=(N // BN,),
        in_specs=[
            pl.BlockSpec((BN, 128), lambda i: (i, 0)),
            pl.BlockSpec((128, 128), lambda i: (0, 0)),
            pl.BlockSpec((1, 128), lambda i: (0, 0)),
        ],
        out_specs=[pl.BlockSpec((BN, 128), lambda i: (i, 0)),
                   pl.BlockSpec((BN, 128), lambda i: (i, 0))],
    )(x, wp, bp)


def _pool_body(h_ref, w_ref, b_ref, o_ref):
    acc = jnp.dot(h_ref[...], w_ref[...], preferred_element_type=jnp.float32)
    o_ref[...] = jnp.maximum(acc + b_ref[...], 0.0)


def _pool(h, w, b):
    dpi, dpo = w.shape
    return pl.pallas_call(
        _pool_body,
        out_shape=jax.ShapeDtypeStruct((N, dpo), jnp.float32),
        grid=(N // BN,),
        in_specs=[
            pl.BlockSpec((BN, dpi), lambda i: (i, 0)),
            pl.BlockSpec((dpi, dpo), lambda i: (0, 0)),
            pl.BlockSpec((1, dpo), lambda i: (0, 0)),
        ],
        out_specs=pl.BlockSpec((BN, dpo), lambda i: (i, 0)),
    )(h, w, b)


def _combine_pool_body(h_ref, a_ref, ws_ref, wn_ref, b_ref, wp_ref, bp_ref,
                       o_ref, hp_ref):
    a = a_ref[...]
    a = jnp.where(a < -1e30, 0.0, a)
    acc = jnp.dot(h_ref[...], ws_ref[...], preferred_element_type=jnp.float32)
    acc += jnp.dot(a, wn_ref[...], preferred_element_type=jnp.float32)
    hn = jnp.maximum(acc + b_ref[...], 0.0)
    o_ref[...] = hn
    acc2 = jnp.dot(hn, wp_ref[...], preferred_element_type=jnp.float32)
    hp_ref[...] = jnp.maximum(acc2 + bp_ref[...], 0.0)


def _combine_pool(h, agg, ws, wn, b, wp, bp):
    dpi = h.shape[1]
    dact = agg.shape[1]
    dpo = ws.shape[1]
    return pl.pallas_call(
        _combine_pool_body,
        out_shape=[jax.ShapeDtypeStruct((N, dpo), jnp.float32),
                   jax.ShapeDtypeStruct((N, 128), jnp.float32)],
        grid=(N // BN,),
        in_specs=[
            pl.BlockSpec((BN, dpi), lambda i: (i, 0)),
            pl.BlockSpec((BN, dact), lambda i: (i, 0)),
            pl.BlockSpec((dpi, dpo), lambda i: (0, 0)),
            pl.BlockSpec((dact, dpo), lambda i: (0, 0)),
            pl.BlockSpec((1, dpo), lambda i: (0, 0)),
            pl.BlockSpec((dpo, 128), lambda i: (0, 0)),
            pl.BlockSpec((1, 128), lambda i: (0, 0)),
        ],
        out_specs=[pl.BlockSpec((BN, dpo), lambda i: (i, 0)),
                   pl.BlockSpec((BN, 128), lambda i: (i, 0))],
    )(h, agg, ws, wn, b, wp, bp)


def _combine_body(h_ref, a_ref, ws_ref, wn_ref, b_ref, o_ref, *, relu):
    a = a_ref[...]
    a = jnp.where(a < -1e30, 0.0, a)
    acc = jnp.dot(h_ref[...], ws_ref[...], preferred_element_type=jnp.float32)
    acc += jnp.dot(a, wn_ref[...], preferred_element_type=jnp.float32)
    acc += b_ref[...]
    if relu:
        acc = jnp.maximum(acc, 0.0)
    o_ref[...] = acc


def _combine(h, agg, ws, wn, b, relu):
    dpi = h.shape[1]
    dact = agg.shape[1]
    dpo = ws.shape[1]
    return pl.pallas_call(
        functools.partial(_combine_body, relu=relu),
        out_shape=jax.ShapeDtypeStruct((N, dpo), jnp.float32),
        grid=(N // BN,),
        in_specs=[
            pl.BlockSpec((BN, dpi), lambda i: (i, 0)),
            pl.BlockSpec((BN, dact), lambda i: (i, 0)),
            pl.BlockSpec((dpi, dpo), lambda i: (0, 0)),
            pl.BlockSpec((dact, dpo), lambda i: (0, 0)),
            pl.BlockSpec((1, dpo), lambda i: (0, 0)),
        ],
        out_specs=pl.BlockSpec((BN, dpo), lambda i: (i, 0)),
    )(h, agg, ws, wn, b)


_MESH = plsc.VectorSubcoreMesh(
    core_axis_name="c", subcore_axis_name="s", num_cores=NC, num_subcores=NS)
_SC_PARAMS = pltpu.CompilerParams(needs_layout_passes=False)


def _wid():
    return lax.axis_index("s") * NC + lax.axis_index("c")


NCHUNK = E // CHUNK


def _partition_body(adj_hbm, part_hbm, cnt_hbm,
                    ad0_v, ad1_v, sel_p, cnt_v, sem0, sem1):
    wid = _wid()
    lo = wid * NPW
    hi = lo + NPW
    ad_v = (ad0_v, ad1_v)
    sems = (sem0, sem1)

    for i in range(CAP // L):
        sel_p[pl.ds(i * L, L)] = jnp.zeros((L,), jnp.int32)

    def grp_body(ad, g, carry):
        nselv, nsel, nfired = carry
        base = g * 4 * L
        s4 = [ad[0, pl.ds(base + k * L, L)] for k in range(4)]
        d4 = [ad[1, pl.ds(base + k * L, L)] for k in range(4)]
        m4 = [(d >= lo) & (d < hi) for d in d4]
        cum4 = [plsc.cumsum(m.astype(jnp.int32)) for m in m4]
        pc4 = [plsc.all_reduce_population_count(m) for m in m4]
        pk4 = [s | ((d - lo) << 16) for s, d in zip(s4, d4)]
        off = nselv - 1
        for k in range(4):
            plsc.store_scatter(sel_p, [off + cum4[k]], pk4[k], mask=m4[k])
            off = off + pc4[k]
        tot = pc4[0] + pc4[1] + pc4[2] + pc4[3]
        nselv = nselv + tot
        nsel = nsel + tot[0]

        full = nsel >= FIRE

        @pl.when(full)
        def _fire():
            pltpu.sync_copy(sel_p.at[pl.ds(0, FIRE)],
                            part_hbm.at[wid, pl.ds(nfired * FIRE, FIRE)])
            for k in range(4):
                sel_p[pl.ds(k * L, L)] = sel_p[pl.ds(FIRE + k * L, L)]

        nselv = jnp.where(full, nselv - FIRE, nselv)
        nsel = jnp.where(full, nsel - FIRE, nsel)
        nfired = jnp.where(full, nfired + 1, nfired)
        return nselv, nsel, nfired

    pltpu.async_copy(adj_hbm.at[:, pl.ds(0, CHUNK)], ad0_v, sem0)
    pltpu.async_copy(adj_hbm.at[:, pl.ds(CHUNK, CHUNK)], ad1_v, sem1)

    def pair_body(i, carry):
        for p in range(2):
            b = 2 * i + p
            pltpu.make_async_copy(
                adj_hbm.at[:, pl.ds(b * CHUNK, CHUNK)], ad_v[p], sems[p]
            ).wait()
            carry = lax.fori_loop(
                0, CHUNK // (4 * L), functools.partial(grp_body, ad_v[p]),
                carry)

            @pl.when(b + 2 < NCHUNK)
            def _refill():
                pltpu.async_copy(
                    adj_hbm.at[:, pl.ds((b + 2) * CHUNK, CHUNK)],
                    ad_v[p], sems[p])
        return carry

    assert NCHUNK % 2 == 0
    _, nsel, nfired = lax.fori_loop(
        0, NCHUNK // 2, pair_body, (jnp.zeros((L,), jnp.int32), 0, 0))

    for j in range(FIRE // G):
        @pl.when(nsel > j * G)
        def _final(j=j):
            pltpu.sync_copy(
                sel_p.at[pl.ds(j * G, G)],
                part_hbm.at[wid, pl.ds(nfired * FIRE + j * G, G)])

    count = nfired * FIRE + nsel
    cnt_v[pl.ds(0, L)] = jnp.full((L,), 1, jnp.int32) * count
    pltpu.sync_copy(cnt_v, cnt_hbm.at[wid])


_partition = pl.kernel(
    _partition_body,
    out_type=[
        jax.ShapeDtypeStruct((NW, E), jnp.int32),
        jax.ShapeDtypeStruct((NW, L), jnp.int32),
    ],
    mesh=_MESH,
    compiler_params=_SC_PARAMS,
    scratch_types=[
        pltpu.VMEM((2, CHUNK), jnp.int32),
        pltpu.VMEM((2, CHUNK), jnp.int32),
        pltpu.VMEM((CAP,), jnp.int32),
        pltpu.VMEM((L,), jnp.int32),
        pltpu.SemaphoreType.DMA,
        pltpu.SemaphoreType.DMA,
    ],
)


def _segmax_body(hp_hbm, part_hbm, cnt_hbm, out_hbm,
                 agg_v, pk0_v, pk1_v, sb0_v, sb1_v, r0_v, r1_v, cnt_v,
                 sp0, sp1, sg0, sg1, *, dact):
    wid = _wid()
    pk = (pk0_v, pk1_v)
    sb = (sb0_v, sb1_v)
    rows = (r0_v, r1_v)
    sp = (sp0, sp1)
    sg = (sg0, sg1)

    pltpu.sync_copy(cnt_hbm.at[wid], cnt_v)
    count = cnt_v[pl.ds(0, L)][0]
    nb = (count + G - 1) // G

    neg = jnp.full((L,), NEG, jnp.float32)

    def init_body(r, _):
        for k in range(dact // L):
            agg_v[r, pl.ds(k * L, L)] = neg
        return 0
    lax.fori_loop(0, NPW, init_body, 0)

    def unpack(p):
        for k in range(G // L):
            sb[p][pl.ds(k * L, L)] = pk[p][pl.ds(k * L, L)] & 0xFFFF

    def start_load(b, p):
        pltpu.async_copy(part_hbm.at[wid, pl.ds(b * G, G)],
                         pk[p].at[pl.ds(0, G)], sp[p])

    def wait_load(b, p):
        pltpu.make_async_copy(part_hbm.at[wid, pl.ds(b * G, G)],
                              pk[p].at[pl.ds(0, G)], sp[p]).wait()

    def start_gather(p):
        pltpu.async_copy(hp_hbm.at[sb[p]], rows[p], sg[p])

    def wait_gather(p):
        pltpu.make_async_copy(hp_hbm.at[sb[p]], rows[p], sg[p]).wait()

    def _upd(p, j, loc):
        nblk = dact // L
        rv = [rows[p][j, pl.ds(k * L, L)] for k in range(nblk)]
        av = [agg_v[loc, pl.ds(k * L, L)] for k in range(nblk)]
        for k in range(nblk):
            agg_v[loc, pl.ds(k * L, L)] = jnp.maximum(av[k], rv[k])

    def drain(p, nd):
        @pl.when(nd == G)
        def _full():
            def blk_body(blk, _):
                jb = blk * L
                locv = pk[p][pl.ds(jb, L)] >> 16
                for lane in range(L):
                    _upd(p, jb + lane, locv[lane])
                return 0
            lax.fori_loop(0, G // L, blk_body, 0)

        @pl.when(nd < G)
        def _partial():
            def edge_body(j, _):
                pval = pk[p][pl.ds(j, L)][0]
                _upd(p, j, pval >> 16)
                return 0
            lax.fori_loop(0, nd, edge_body, 0)

    @pl.when(nb > 0)
    def _prologue():
        pltpu.sync_copy(part_hbm.at[wid, pl.ds(0, G)], pk[0].at[pl.ds(0, G)])
        unpack(0)
        start_gather(0)

        @pl.when(nb > 1)
        def _():
            start_load(1, 1)

    def pair_body(i, _):
        for p in range(2):
            b = 2 * i + p
            q = 1 - p

            @pl.when(b < nb)
            def _do():
                @pl.when(b + 1 < nb)
                def _():
                    wait_load(b + 1, q)
                    unpack(q)
                    start_gather(q)

                wait_gather(p)
                drain(p, jnp.minimum(G, count - b * G))

                @pl.when(b + 2 < nb)
                def _():
                    start_load(b + 2, p)
        return 0

    lax.fori_loop(0, (nb + 1) // 2, pair_body, 0)
    pltpu.sync_copy(agg_v, out_hbm.at[pl.ds(wid * NPW, NPW)])


@functools.lru_cache(maxsize=None)
def _make_segmax(dact):
    return pl.kernel(
        functools.partial(_segmax_body, dact=dact),
        out_type=jax.ShapeDtypeStruct((NPAD, dact), jnp.float32),
        mesh=_MESH,
        compiler_params=_SC_PARAMS,
        scratch_types=[
            pltpu.VMEM((NPW, dact), jnp.float32),
            pltpu.VMEM((G + L,), jnp.int32),
            pltpu.VMEM((G + L,), jnp.int32),
            pltpu.VMEM((G,), jnp.int32),
            pltpu.VMEM((G,), jnp.int32),
            pltpu.VMEM((G, 128), jnp.float32),
            pltpu.VMEM((G, 128), jnp.float32),
            pltpu.VMEM((L,), jnp.int32),
            pltpu.SemaphoreType.DMA,
            pltpu.SemaphoreType.DMA,
            pltpu.SemaphoreType.DMA,
            pltpu.SemaphoreType.DMA,
        ],
    )


def _pad2(w, r, c):
    return jnp.pad(w, ((0, r - w.shape[0]), (0, c - w.shape[1])))


def kernel(x, adj, params):
    dpi_l = [128, 80, 80, 80, 80]
    dact_l = [128, 80, 80, 80, 80]
    dpo_l = [80, 80, 80, 80, 128]

    wp_l = [_pad2(params['Wp%d' % i], dpi_l[i], 128) for i in range(5)]
    bp_l = [jnp.pad(params['bp%d' % i],
                    (0, 128 - params['bp%d' % i].shape[0])).reshape(1, 128)
            for i in range(5)]

    part, cnts = _partition(adj)
    h, hp = _norm_pool(x, wp_l[0], bp_l[0])

    for i in range(5):
        dpi, dact, dpo = dpi_l[i], dact_l[i], dpo_l[i]
        ws = _pad2(params['Ws%d' % i], dpi, dpo)
        wn = _pad2(params['Wn%d' % i], dact, dpo)
        b = jnp.pad(params['b%d' % i],
                    (0, dpo - params['b%d' % i].shape[0])).reshape(1, dpo)

        agg = _make_segmax(dact)(hp, part, cnts)[:N]
        if i < 4:
            h, hp = _combine_pool(h, agg, ws, wn, b, wp_l[i + 1], bp_l[i + 1])
        else:
            h = _combine(h, agg, ws, wn, b, relu=False)

    return h


# speedup vs baseline: 8.0227x; 1.0153x over previous
import functools

import jax
import jax.numpy as jnp
from jax import lax
from jax.experimental import pallas as pl
from jax.experimental.pallas import tpu as pltpu
from jax.experimental.pallas import tpu_sc as plsc

N = 10000
E = 320000
NC, NS, L = 2, 16, 16
NW = NC * NS
NPW = 320
NPAD = NW * NPW
G = 128
FIRE = 3 * G
NGRP = 8
CAP = FIRE + NGRP * L
CHUNK = 6400
NEG = -3.0e38

BN = 1000


def _norm_pool_body(x_ref, wp_ref, bp_ref, h_ref, hp_ref):
    x = x_ref[...]
    ss = jnp.sum(x * x, axis=1, keepdims=True)
    nrm = jnp.maximum(jnp.sqrt(ss), 1e-12)
    h = x / nrm
    h_ref[...] = h
    acc = jnp.dot(h, wp_ref[...], preferred_element_type=jnp.float32)
    hp_ref[...] = jnp.maximum(acc + bp_ref[...], 0.0)


def _norm_pool(x, wp, bp):
    return pl.pallas_call(
        _norm_pool_body,
        out_shape=[jax.ShapeDtypeStruct((N, 128), jnp.float32),
                   jax.ShapeDtypeStruct((N, 128), jnp.float32)],
        grid=(N // BN,),
        in_specs=[
            pl.BlockSpec((BN, 128), lambda i: (i, 0)),
            pl.BlockSpec((128, 128), lambda i: (0, 0)),
            pl.BlockSpec((1, 128), lambda i: (0, 0)),
        ],
        out_specs=[pl.BlockSpec((BN, 128), lambda i: (i, 0)),
                   pl.BlockSpec((BN, 128), lambda i: (i, 0))],
    )(x, wp, bp)


def _pool_body(h_ref, w_ref, b_ref, o_ref):
    acc = jnp.dot(h_ref[...], w_ref[...], preferred_element_type=jnp.float32)
    o_ref[...] = jnp.maximum(acc + b_ref[...], 0.0)


def _pool(h, w, b):
    dpi, dpo = w.shape
    return pl.pallas_call(
        _pool_body,
        out_shape=jax.ShapeDtypeStruct((N, dpo), jnp.float32),
        grid=(N // BN,),
        in_specs=[
            pl.BlockSpec((BN, dpi), lambda i: (i, 0)),
            pl.BlockSpec((dpi, dpo), lambda i: (0, 0)),
            pl.BlockSpec((1, dpo), lambda i: (0, 0)),
        ],
        out_specs=pl.BlockSpec((BN, dpo), lambda i: (i, 0)),
    )(h, w, b)


def _combine_pool_body(h_ref, a_ref, ws_ref, wn_ref, b_ref, wp_ref, bp_ref,
                       o_ref, hp_ref):
    a = a_ref[...]
    a = jnp.where(a < -1e30, 0.0, a)
    acc = jnp.dot(h_ref[...], ws_ref[...], preferred_element_type=jnp.float32)
    acc += jnp.dot(a, wn_ref[...], preferred_element_type=jnp.float32)
    hn = jnp.maximum(acc + b_ref[...], 0.0)
    o_ref[...] = hn
    acc2 = jnp.dot(hn, wp_ref[...], preferred_element_type=jnp.float32)
    hp_ref[...] = jnp.maximum(acc2 + bp_ref[...], 0.0)


def _combine_pool(h, agg, ws, wn, b, wp, bp):
    dpi = h.shape[1]
    dact = agg.shape[1]
    dpo = ws.shape[1]
    return pl.pallas_call(
        _combine_pool_body,
        out_shape=[jax.ShapeDtypeStruct((N, dpo), jnp.float32),
                   jax.ShapeDtypeStruct((N, 128), jnp.float32)],
        grid=(N // BN,),
        in_specs=[
            pl.BlockSpec((BN, dpi), lambda i: (i, 0)),
            pl.BlockSpec((BN, dact), lambda i: (i, 0)),
            pl.BlockSpec((dpi, dpo), lambda i: (0, 0)),
            pl.BlockSpec((dact, dpo), lambda i: (0, 0)),
            pl.BlockSpec((1, dpo), lambda i: (0, 0)),
            pl.BlockSpec((dpo, 128), lambda i: (0, 0)),
            pl.BlockSpec((1, 128), lambda i: (0, 0)),
        ],
        out_specs=[pl.BlockSpec((BN, dpo), lambda i: (i, 0)),
                   pl.BlockSpec((BN, 128), lambda i: (i, 0))],
    )(h, agg, ws, wn, b, wp, bp)


def _combine_body(h_ref, a_ref, ws_ref, wn_ref, b_ref, o_ref, *, relu):
    a = a_ref[...]
    a = jnp.where(a < -1e30, 0.0, a)
    acc = jnp.dot(h_ref[...], ws_ref[...], preferred_element_type=jnp.float32)
    acc += jnp.dot(a, wn_ref[...], preferred_element_type=jnp.float32)
    acc += b_ref[...]
    if relu:
        acc = jnp.maximum(acc, 0.0)
    o_ref[...] = acc


def _combine(h, agg, ws, wn, b, relu):
    dpi = h.shape[1]
    dact = agg.shape[1]
    dpo = ws.shape[1]
    return pl.pallas_call(
        functools.partial(_combine_body, relu=relu),
        out_shape=jax.ShapeDtypeStruct((N, dpo), jnp.float32),
        grid=(N // BN,),
        in_specs=[
            pl.BlockSpec((BN, dpi), lambda i: (i, 0)),
            pl.BlockSpec((BN, dact), lambda i: (i, 0)),
            pl.BlockSpec((dpi, dpo), lambda i: (0, 0)),
            pl.BlockSpec((dact, dpo), lambda i: (0, 0)),
            pl.BlockSpec((1, dpo), lambda i: (0, 0)),
        ],
        out_specs=pl.BlockSpec((BN, dpo), lambda i: (i, 0)),
    )(h, agg, ws, wn, b)


_MESH = plsc.VectorSubcoreMesh(
    core_axis_name="c", subcore_axis_name="s", num_cores=NC, num_subcores=NS)
_SC_PARAMS = pltpu.CompilerParams(needs_layout_passes=False)


def _wid():
    return lax.axis_index("s") * NC + lax.axis_index("c")


NCHUNK = E // CHUNK


def _partition_body(adj_hbm, part_hbm, cnt_hbm,
                    ad0_v, ad1_v, sel_p, cnt_v, sem0, sem1):
    wid = _wid()
    lo = wid * NPW
    hi = lo + NPW
    ad_v = (ad0_v, ad1_v)
    sems = (sem0, sem1)

    for i in range(CAP // L):
        sel_p[pl.ds(i * L, L)] = jnp.zeros((L,), jnp.int32)

    def grp_body(ad, g, carry):
        nselv, nsel, nfired = carry
        base = g * NGRP * L
        s4 = [ad[0, pl.ds(base + k * L, L)] for k in range(NGRP)]
        d4 = [ad[1, pl.ds(base + k * L, L)] for k in range(NGRP)]
        m4 = [(d >= lo) & (d < hi) for d in d4]
        cum4 = [plsc.cumsum(m.astype(jnp.int32)) for m in m4]
        pc4 = [plsc.all_reduce_population_count(m) for m in m4]
        pk4 = [s | ((d - lo) << 16) for s, d in zip(s4, d4)]
        off = nselv - 1
        for k in range(NGRP):
            plsc.store_scatter(sel_p, [off + cum4[k]], pk4[k], mask=m4[k])
            off = off + pc4[k]
        tot = pc4[0]
        for k in range(1, NGRP):
            tot = tot + pc4[k]
        nselv = nselv + tot
        nsel = nsel + tot[0]

        full = nsel >= FIRE

        @pl.when(full)
        def _fire():
            pltpu.sync_copy(sel_p.at[pl.ds(0, FIRE)],
                            part_hbm.at[wid, pl.ds(nfired * FIRE, FIRE)])
            for k in range(NGRP):
                sel_p[pl.ds(k * L, L)] = sel_p[pl.ds(FIRE + k * L, L)]

        nselv = jnp.where(full, nselv - FIRE, nselv)
        nsel = jnp.where(full, nsel - FIRE, nsel)
        nfired = jnp.where(full, nfired + 1, nfired)
        return nselv, nsel, nfired

    pltpu.async_copy(adj_hbm.at[:, pl.ds(0, CHUNK)], ad0_v, sem0)
    pltpu.async_copy(adj_hbm.at[:, pl.ds(CHUNK, CHUNK)], ad1_v, sem1)

    def pair_body(i, carry):
        for p in range(2):
            b = 2 * i + p
            pltpu.make_async_copy(
                adj_hbm.at[:, pl.ds(b * CHUNK, CHUNK)], ad_v[p], sems[p]
            ).wait()
            carry = lax.fori_loop(
                0, CHUNK // (NGRP * L), functools.partial(grp_body, ad_v[p]),
                carry)

            @pl.when(b + 2 < NCHUNK)
            def _refill():
                pltpu.async_copy(
                    adj_hbm.at[:, pl.ds((b + 2) * CHUNK, CHUNK)],
                    ad_v[p], sems[p])
        return carry

    assert NCHUNK % 2 == 0
    _, nsel, nfired = lax.fori_loop(
        0, NCHUNK // 2, pair_body, (jnp.zeros((L,), jnp.int32), 0, 0))

    for j in range(FIRE // G):
        @pl.when(nsel > j * G)
        def _final(j=j):
            pltpu.sync_copy(
                sel_p.at[pl.ds(j * G, G)],
                part_hbm.at[wid, pl.ds(nfired * FIRE + j * G, G)])

    count = nfired * FIRE + nsel
    cnt_v[pl.ds(0, L)] = jnp.full((L,), 1, jnp.int32) * count
    pltpu.sync_copy(cnt_v, cnt_hbm.at[wid])


_partition = pl.kernel(
    _partition_body,
    out_type=[
        jax.ShapeDtypeStruct((NW, E), jnp.int32),
        jax.ShapeDtypeStruct((NW, L), jnp.int32),
    ],
    mesh=_MESH,
    compiler_params=_SC_PARAMS,
    scratch_types=[
        pltpu.VMEM((2, CHUNK), jnp.int32),
        pltpu.VMEM((2, CHUNK), jnp.int32),
        pltpu.VMEM((CAP,), jnp.int32),
        pltpu.VMEM((L,), jnp.int32),
        pltpu.SemaphoreType.DMA,
        pltpu.SemaphoreType.DMA,
    ],
)


def _segmax_body(hp_hbm, part_hbm, cnt_hbm, out_hbm,
                 agg_v, pk0_v, pk1_v, sb0_v, sb1_v, r0_v, r1_v, cnt_v,
                 sp0, sp1, sg0, sg1, *, dact):
    wid = _wid()
    pk = (pk0_v, pk1_v)
    sb = (sb0_v, sb1_v)
    rows = (r0_v, r1_v)
    sp = (sp0, sp1)
    sg = (sg0, sg1)

    pltpu.sync_copy(cnt_hbm.at[wid], cnt_v)
    count = cnt_v[pl.ds(0, L)][0]
    nb = (count + G - 1) // G

    neg = jnp.full((L,), NEG, jnp.float32)

    def init_body(r, _):
        for k in range(dact // L):
            agg_v[r, pl.ds(k * L, L)] = neg
        return 0
    lax.fori_loop(0, NPW, init_body, 0)

    def unpack(p):
        for k in range(G // L):
            sb[p][pl.ds(k * L, L)] = pk[p][pl.ds(k * L, L)] & 0xFFFF

    def start_load(b, p):
        pltpu.async_copy(part_hbm.at[wid, pl.ds(b * G, G)],
                         pk[p].at[pl.ds(0, G)], sp[p])

    def wait_load(b, p):
        pltpu.make_async_copy(part_hbm.at[wid, pl.ds(b * G, G)],
                              pk[p].at[pl.ds(0, G)], sp[p]).wait()

    def start_gather(p):
        pltpu.async_copy(hp_hbm.at[sb[p]], rows[p], sg[p])

    def wait_gather(p):
        pltpu.make_async_copy(hp_hbm.at[sb[p]], rows[p], sg[p]).wait()

    def _upd(p, j, loc):
        nblk = dact // L
        rv = [rows[p][j, pl.ds(k * L, L)] for k in range(nblk)]
        av = [agg_v[loc, pl.ds(k * L, L)] for k in range(nblk)]
        for k in range(nblk):
            agg_v[loc, pl.ds(k * L, L)] = jnp.maximum(av[k], rv[k])

    def drain(p, nd):
        @pl.when(nd == G)
        def _full():
            def blk_body(blk, _):
                jb = blk * L
                locv = pk[p][pl.ds(jb, L)] >> 16
                for lane in range(L):
                    _upd(p, jb + lane, locv[lane])
                return 0
            lax.fori_loop(0, G // L, blk_body, 0)

        @pl.when(nd < G)
        def _partial():
            def edge_body(j, _):
                pval = pk[p][pl.ds(j, L)][0]
                _upd(p, j, pval >> 16)
                return 0
            lax.fori_loop(0, nd, edge_body, 0)

    @pl.when(nb > 0)
    def _prologue():
        pltpu.sync_copy(part_hbm.at[wid, pl.ds(0, G)], pk[0].at[pl.ds(0, G)])
        unpack(0)
        start_gather(0)

        @pl.when(nb > 1)
        def _():
            start_load(1, 1)

    def pair_body(i, _):
        for p in range(2):
            b = 2 * i + p
            q = 1 - p

            @pl.when(b < nb)
            def _do():
                @pl.when(b + 1 < nb)
                def _():
                    wait_load(b + 1, q)
                    unpack(q)
                    start_gather(q)

                wait_gather(p)
                drain(p, jnp.minimum(G, count - b * G))

                @pl.when(b + 2 < nb)
                def _():
                    start_load(b + 2, p)
        return 0

    lax.fori_loop(0, (nb + 1) // 2, pair_body, 0)
    pltpu.sync_copy(agg_v, out_hbm.at[pl.ds(wid * NPW, NPW)])


@functools.lru_cache(maxsize=None)
def _make_segmax(dact):
    return pl.kernel(
        functools.partial(_segmax_body, dact=dact),
        out_type=jax.ShapeDtypeStruct((NPAD, dact), jnp.float32),
        mesh=_MESH,
        compiler_params=_SC_PARAMS,
        scratch_types=[
            pltpu.VMEM((NPW, dact), jnp.float32),
            pltpu.VMEM((G + L,), jnp.int32),
            pltpu.VMEM((G + L,), jnp.int32),
            pltpu.VMEM((G,), jnp.int32),
            pltpu.VMEM((G,), jnp.int32),
            pltpu.VMEM((G, 128), jnp.float32),
            pltpu.VMEM((G, 128), jnp.float32),
            pltpu.VMEM((L,), jnp.int32),
            pltpu.SemaphoreType.DMA,
            pltpu.SemaphoreType.DMA,
            pltpu.SemaphoreType.DMA,
            pltpu.SemaphoreType.DMA,
        ],
    )


def _pad2(w, r, c):
    return jnp.pad(w, ((0, r - w.shape[0]), (0, c - w.shape[1])))


def kernel(x, adj, params):
    dpi_l = [128, 80, 80, 80, 80]
    dact_l = [128, 80, 80, 80, 80]
    dpo_l = [80, 80, 80, 80, 128]

    wp_l = [_pad2(params['Wp%d' % i], dpi_l[i], 128) for i in range(5)]
    bp_l = [jnp.pad(params['bp%d' % i],
                    (0, 128 - params['bp%d' % i].shape[0])).reshape(1, 128)
            for i in range(5)]

    part, cnts = _partition(adj)
    h, hp = _norm_pool(x, wp_l[0], bp_l[0])

    for i in range(5):
        dpi, dact, dpo = dpi_l[i], dact_l[i], dpo_l[i]
        ws = _pad2(params['Ws%d' % i], dpi, dpo)
        wn = _pad2(params['Wn%d' % i], dact, dpo)
        b = jnp.pad(params['b%d' % i],
                    (0, dpo - params['b%d' % i].shape[0])).reshape(1, dpo)

        agg = _make_segmax(dact)(hp, part, cnts)[:N]
        if i < 4:
            h, hp = _combine_pool(h, agg, ws, wn, b, wp_l[i + 1], bp_l[i + 1])
        else:
            h = _combine(h, agg, ws, wn, b, relu=False)

    return h
